# Initial kernel scaffold; baseline (speedup 1.0000x reference)
import jax
import jax.numpy as jnp
from jax.experimental import pallas as pl


def kernel(x, pos, edge_index, batch, params):
    raise NotImplementedError("write your pallas kernel here")


# speedup vs baseline: 1.7871x; 1.7871x over previous
import functools

import jax
import jax.numpy as jnp
import numpy as np
from jax.experimental import pallas as pl
from jax.experimental.pallas import tpu as pltpu

N = 10000
E = 160000
H = 256
NUM_GRAPHS = 16
RADIUS = 4.0
SCALE_FACTORS = (1.0, 2.0, 4.0)
SCALE_DIMS = (86, 85, 85)

_SEG_NP = np.zeros((3, 256), np.float32)
_SEG_NP[0, :86] = 1.0
_SEG_NP[1, 86:171] = 1.0
_SEG_NP[2, 171:256] = 1.0
_DIMS_NP = np.asarray([86.0, 85.0, 85.0], np.float32)

_NB = 1000
_EB = 2000


def _silu(x):
    return x * jax.nn.sigmoid(x)


def _ln(x, g, b):
    mu = jnp.mean(x, axis=-1, keepdims=True)
    var = jnp.mean((x - mu) ** 2, axis=-1, keepdims=True)
    return g * (x - mu) * jax.lax.rsqrt(var + 1e-5) + b


def _row(v):
    return v.reshape(1, -1)


def _pre_body(x_ref, pos_ref,
              win, bin_, gin, bln,
              p1a, p1ab, p1b, p1bb, p2a, p2ab, p2b, p2bb,
              wa1, wb1,
              xe1_o, a1_o, b1_o, pe2_o):
    x = x_ref[...]
    pos = pos_ref[...]
    h = _silu(_ln(jnp.dot(x, win[...], preferred_element_type=jnp.float32)
                  + bin_[...], gin[...], bln[...]))
    pe1 = jnp.dot(_silu(jnp.dot(pos, p1a[...], preferred_element_type=jnp.float32)
                        + p1ab[...]), p1b[...],
                  preferred_element_type=jnp.float32) + p1bb[...]
    pe2 = jnp.dot(_silu(jnp.dot(pos, p2a[...], preferred_element_type=jnp.float32)
                        + p2ab[...]), p2b[...],
                  preferred_element_type=jnp.float32) + p2bb[...]
    xe1 = h + pe1
    xe1_o[...] = xe1
    pe2_o[...] = pe2
    a1_o[...] = jnp.dot(xe1, wa1[...], preferred_element_type=jnp.float32)
    b1_o[...] = jnp.dot(xe1, wb1[...], preferred_element_type=jnp.float32)


def _k_pre(x, pos8, W):
    grid = (N // _NB,)
    bspec_n = pl.BlockSpec((_NB, 256), lambda i: (i, 0))
    bspec_p = pl.BlockSpec((_NB, 8), lambda i: (i, 0))
    cw = lambda shape: pl.BlockSpec(shape, lambda i: tuple(0 for _ in shape))
    out_shapes = [jax.ShapeDtypeStruct((N, 256), jnp.float32)] * 4
    return pl.pallas_call(
        _pre_body,
        grid=grid,
        in_specs=[bspec_n, bspec_p,
                  cw((256, 256)), cw((1, 256)), cw((1, 256)), cw((1, 256)),
                  cw((8, 128)), cw((1, 128)), cw((128, 256)), cw((1, 256)),
                  cw((8, 128)), cw((1, 128)), cw((128, 256)), cw((1, 256)),
                  cw((256, 256)), cw((256, 256))],
        out_specs=[bspec_n] * 4,
        out_shape=out_shapes,
    )(x, pos8, *W)


def _edge_body(g_ref, seg, gg, bb, p_o):
    G = g_ref[...]
    segm = seg[...]
    dims = jnp.sum(segm, axis=1, keepdims=True).T
    mu = jnp.dot(G, segm.T, preferred_element_type=jnp.float32) / dims
    muf = jnp.dot(mu, segm, preferred_element_type=jnp.float32)
    cen = G - muf
    var = jnp.dot(cen * cen, segm.T, preferred_element_type=jnp.float32) / dims
    denom = jax.lax.rsqrt(jnp.dot(var, segm, preferred_element_type=jnp.float32) + 1e-5)
    y = gg[...] * cen * denom + bb[...]
    p_o[...] = _silu(y)


def _k_edge(G, seg, gcat, bcat):
    e = G.shape[0]
    grid = (e // _EB,)
    bspec = pl.BlockSpec((_EB, 256), lambda i: (i, 0))
    cw = lambda shape: pl.BlockSpec(shape, lambda i: tuple(0 for _ in shape))
    return pl.pallas_call(
        _edge_body,
        grid=grid,
        in_specs=[bspec, cw((3, 256)), cw((1, 256)), cw((1, 256))],
        out_specs=bspec,
        out_shape=jax.ShapeDtypeStruct((e, 256), jnp.float32),
    )(G, seg, gcat, bcat)


def _node_body(has_next, s_ref, deg_ref, xe_ref, pen_ref,
               w2bd, b2, seg, ws1, bs1, ws2, bs2,
               wu1x, wu1w, bu1, gu, bu, wu2, bu2, gn, bn,
               wan, wbn, o0, o1, o2):
    S = s_ref[...]
    xe = xe_ref[...]
    agg = (jnp.dot(S, w2bd[...], preferred_element_type=jnp.float32)
           + deg_ref[...] * b2[...])
    t = _silu(jnp.dot(agg, ws1[...], preferred_element_type=jnp.float32) + bs1[...])
    logit = jnp.dot(t, ws2[...], preferred_element_type=jnp.float32) + bs2[...]
    aw = jax.nn.softmax(logit, axis=-1)
    weighted = agg * jnp.dot(aw, seg[...], preferred_element_type=jnp.float32)
    u = (jnp.dot(xe, wu1x[...], preferred_element_type=jnp.float32)
         + jnp.dot(weighted, wu1w[...], preferred_element_type=jnp.float32)
         + bu1[...])
    u = _silu(_ln(u, gu[...], bu[...]))
    u = jnp.dot(u, wu2[...], preferred_element_type=jnp.float32) + bu2[...]
    h = _ln(u + xe, gn[...], bn[...])
    if has_next:
        xe2 = h + pen_ref[...]
        o0[...] = xe2
        o1[...] = jnp.dot(xe2, wan[...], preferred_element_type=jnp.float32)
        o2[...] = jnp.dot(xe2, wbn[...], preferred_element_type=jnp.float32)
    else:
        o0[...] = h


def _k_node(S, deg2d, xe, pe_next, W, has_next):
    grid = (N // _NB,)
    bspec_n = pl.BlockSpec((_NB, 256), lambda i: (i, 0))
    bspec_d = pl.BlockSpec((_NB, 1), lambda i: (i, 0))
    cw = lambda shape: pl.BlockSpec(shape, lambda i: tuple(0 for _ in shape))
    w_specs = [cw((256, 256)), cw((1, 256)), cw((3, 256)),
               cw((256, 64)), cw((1, 64)), cw((64, 3)), cw((1, 3)),
               cw((256, 512)), cw((256, 512)), cw((1, 512)), cw((1, 512)), cw((1, 512)),
               cw((512, 256)), cw((1, 256)), cw((1, 256)), cw((1, 256)),
               cw((256, 256)), cw((256, 256))]
    if has_next:
        out_specs = [bspec_n] * 3
        out_shape = [jax.ShapeDtypeStruct((N, 256), jnp.float32)] * 3
    else:
        out_specs = [bspec_n]
        out_shape = [jax.ShapeDtypeStruct((N, 256), jnp.float32)]
    outs = pl.pallas_call(
        _wrap_node_body(has_next),
        grid=grid,
        in_specs=[bspec_n, bspec_d, bspec_n, bspec_n] + w_specs,
        out_specs=out_specs,
        out_shape=out_shape,
    )(S, deg2d, xe, pe_next, *W)
    return outs


def _wrap_node_body(has_next):
    if has_next:
        def b(s, d, xe, pen, *rest):
            *ws, o0, o1, o2 = rest
            _node_body(True, s, d, xe, pen, *ws, o0, o1, o2)
    else:
        def b(s, d, xe, pen, *rest):
            *ws, o0 = rest
            _node_body(False, s, d, xe, pen, *ws, o0, None, None)
    return b


def _layer_consts(lp):
    WAs, WBs, Wcs, c0s, gc, bc, b2c = [], [], [], [], [], [], []
    blocks = []
    for i, s in enumerate(SCALE_FACTORS):
        mp = lp['msg'][i]
        W1 = mp['l1']['w']
        WAs.append(W1[:256])
        WBs.append(W1[256:512])
        Wcs.append(W1[512:516])
        c0s.append(s * W1[516] + mp['l1']['b'])
        gc.append(mp['ln']['g'])
        bc.append(mp['ln']['b'])
        blocks.append(mp['l2']['w'])
        b2c.append(mp['l2']['b'])
    W2bd = jax.scipy.linalg.block_diag(*blocks)
    return dict(
        WA=jnp.concatenate(WAs, 1), WB=jnp.concatenate(WBs, 1),
        Wc=jnp.concatenate(Wcs, 1), c0=jnp.concatenate(c0s),
        g=jnp.concatenate(gc), b=jnp.concatenate(bc),
        W2bd=W2bd, b2=jnp.concatenate(b2c))


def kernel(x, pos, edge_index, batch, params):
    seg = jnp.asarray(_SEG_NP)
    row, col = edge_index[0], edge_index[1]

    L1, L2 = params['layers'][0], params['layers'][1]
    C1, C2 = _layer_consts(L1), _layer_consts(L2)

    ev = pos[row] - pos[col]
    ed = jnp.sqrt(jnp.sum(ev * ev, axis=-1, keepdims=True))
    ea4 = jnp.concatenate([ev / (ed + 1e-8), ed / (RADIUS + 1e-8)], -1)
    deg = jnp.zeros((N,), jnp.float32).at[col].add(1.0)
    deg2d = deg[:, None]

    pos8 = jnp.pad(pos, ((0, 0), (0, 5)))
    ip = params['input_proj']
    pe_w = lambda lp, k: lp['pos_enc'][k]
    pre_W = [
        ip['lin']['w'], _row(ip['lin']['b']), _row(ip['ln']['g']), _row(ip['ln']['b']),
        jnp.pad(pe_w(L1, 'l1')['w'], ((0, 5), (0, 0))), _row(pe_w(L1, 'l1')['b']),
        pe_w(L1, 'l2')['w'], _row(pe_w(L1, 'l2')['b']),
        jnp.pad(pe_w(L2, 'l1')['w'], ((0, 5), (0, 0))), _row(pe_w(L2, 'l1')['b']),
        pe_w(L2, 'l2')['w'], _row(pe_w(L2, 'l2')['b']),
        C1['WA'], C1['WB'],
    ]
    xe1, A1, B1, pe2 = _k_pre(x, pos8, pre_W)

    def edge_phase(A, B, C):
        G = A[col] + B[row] + ea4 @ C['Wc'] + C['c0']
        P = _k_edge(G, seg, _row(C['g']), _row(C['b']))
        S = jnp.zeros((N, 256), jnp.float32).at[col].add(P)
        return S

    def node_W(lp, C, Cn):
        up = lp['update']
        sa = lp['scale_att']
        Wu1 = up['l1']['w']
        return [
            C['W2bd'], _row(C['b2']), seg,
            sa['l1']['w'], _row(sa['l1']['b']), sa['l2']['w'], _row(sa['l2']['b']),
            Wu1[:256], Wu1[256:], _row(up['l1']['b']),
            _row(up['ln']['g']), _row(up['ln']['b']),
            up['l2']['w'], _row(up['l2']['b']),
            _row(lp['norm']['g']), _row(lp['norm']['b']),
            (Cn['WA'] if Cn is not None else C['WA']),
            (Cn['WB'] if Cn is not None else C['WB']),
        ]

    S1 = edge_phase(A1, B1, C1)
    xe2, A2, B2 = _k_node(S1, deg2d, xe1, pe2, node_W(L1, C1, C2), True)
    S2 = edge_phase(A2, B2, C2)
    (h,) = _k_node(S2, deg2d, xe2, pe2, node_W(L2, C2, None), False)

    ones = jnp.ones((N,), jnp.float32)
    cnt = jax.ops.segment_sum(ones, batch, num_segments=NUM_GRAPHS)
    mean_pool = jax.ops.segment_sum(h, batch, num_segments=NUM_GRAPHS) / jnp.maximum(cnt, 1.0)[:, None]
    max_pool = jax.ops.segment_max(h, batch, num_segments=NUM_GRAPHS)
    ap = params['att_pool']
    att = jax.nn.softmax((_silu(h @ ap['l1']['w'] + ap['l1']['b'])) @ ap['l2']['w'] + ap['l2']['b'], axis=0)
    att_pool = jax.ops.segment_sum(h * att, batch, num_segments=NUM_GRAPHS)
    combined = jnp.concatenate([mean_pool, max_pool, att_pool], -1)
    pf = params['pool_fusion']
    pooled = _silu(_ln(combined @ pf['lin']['w'] + pf['lin']['b'], pf['ln']['g'], pf['ln']['b']))
    t = params['topo']

    def head(hp, x_):
        return (_silu(x_ @ hp['l1']['w'] + hp['l1']['b'])) @ hp['l2']['w'] + hp['l2']['b']

    chern = jnp.tanh(head(t['chern'], pooled))
    z2 = jax.nn.sigmoid(head(t['z2'], pooled))
    gap = jax.nn.softplus(head(t['gap'], pooled))
    sym = head(t['sym'], pooled)
    topo_feat = jnp.concatenate([chern, z2, gap, sym], -1)
    f = jnp.concatenate([pooled, topo_feat], -1)
    op = params['output_proj']
    f = _silu(_ln(f @ op['l1']['w'] + op['l1']['b'], op['ln1']['g'], op['ln1']['b']))
    f = _silu(_ln(f @ op['l2']['w'] + op['l2']['b'], op['ln2']['g'], op['ln2']['b']))
    out = f @ op['l3']['w'] + op['l3']['b']
    th = params['topo_head']
    logits = head(th, out)
    return out, logits, topo_feat


# speedup vs baseline: 3.4229x; 1.9153x over previous
import functools

import jax
import jax.numpy as jnp
import numpy as np
from jax import lax
from jax.experimental import pallas as pl
from jax.experimental.pallas import tpu as pltpu
from jax.experimental.pallas import tpu_sc as plsc

N = 10000
E = 160000
H = 256
NUM_GRAPHS = 16
RADIUS = 4.0
SCALE_FACTORS = (1.0, 2.0, 4.0)

_SEG_NP = np.zeros((3, 256), np.float32)
_SEG_NP[0, :86] = 1.0
_SEG_NP[1, 86:171] = 1.0
_SEG_NP[2, 171:256] = 1.0

_NB = 1000
_EB = 2000

_NSUB = 16
_CH = 80
_EPT = E // _NSUB
_NCHUNK = _EPT // _CH
_CHP = 40
_EPT32 = E // 32
_NCHP = _EPT32 // _CHP
_ROWS_PT = 1000
_NDUMP = N // _ROWS_PT

_sc_mesh = lambda: plsc.VectorSubcoreMesh(core_axis_name="c", subcore_axis_name="s")


def _silu(x):
    return x * jax.nn.sigmoid(x)


def _ln(x, g, b):
    mu = jnp.mean(x, axis=-1, keepdims=True)
    var = jnp.mean((x - mu) ** 2, axis=-1, keepdims=True)
    return g * (x - mu) * jax.lax.rsqrt(var + 1e-5) + b


def _row(v):
    return v.reshape(1, -1)


def _sc_prologue(colp, rowp, pos128, zrows):
    @functools.partial(
        pl.kernel,
        mesh=_sc_mesh(),
        out_type=[
            jax.ShapeDtypeStruct((E, 16), jnp.float32),
            jax.ShapeDtypeStruct((2, N, 128), jnp.float32),
        ],
        scratch_types=[
            pltpu.VMEM((_CHP,), jnp.int32),
            pltpu.VMEM((_CHP,), jnp.int32),
            pltpu.VMEM((_CHP, 128), jnp.float32),
            pltpu.VMEM((_CHP, 128), jnp.float32),
            pltpu.VMEM((_CHP, 128), jnp.float32),
            pltpu.VMEM((_CHP, 16), jnp.float32),
            pltpu.VMEM_SHARED((N, 128), jnp.float32),
            pltpu.SemaphoreType.DMA,
            pltpu.SemaphoreType.DMA,
        ],
    )
    def k(col_h, row_h, pos_h, z_h, evec_o, deg_o,
          ci_v, ri_v, pr_v, pc_v, ones_v, vec_v, deg_sh, sem1, sem2):
        c = lax.axis_index("c")
        s = lax.axis_index("s")
        wid = c * _NSUB + s
        base = wid * _EPT32

        def fill(i, _):
            for j in range(8):
                ones_v[i, pl.ds(j * 16, 16)] = jnp.full((16,), 1.0, jnp.float32)
            return 0
        lax.fori_loop(0, _CHP, fill, 0)

        @pl.when(s < _NDUMP)
        def _():
            pltpu.sync_copy(z_h, deg_sh.at[pl.ds(s * _ROWS_PT, _ROWS_PT)])
        plsc.subcore_barrier()

        def chunk(kk, _):
            off = base + kk * _CHP
            pltpu.sync_copy(col_h.at[pl.ds(off, _CHP)], ci_v)
            pltpu.sync_copy(row_h.at[pl.ds(off, _CHP)], ri_v)
            cp1 = pltpu.async_copy(pos_h.at[ri_v], pr_v, sem1)
            cp2 = pltpu.async_copy(pos_h.at[ci_v], pc_v, sem2)
            cp1.wait()
            cp2.wait()

            def sub(i, _):
                vec_v[i, pl.ds(0, 16)] = (pr_v[i, pl.ds(0, 16)]
                                          - pc_v[i, pl.ds(0, 16)])
                return 0
            lax.fori_loop(0, _CHP, sub, 0)
            pltpu.sync_copy(vec_v, evec_o.at[pl.ds(off, _CHP)])
            pltpu.sync_copy(ones_v, deg_sh.at[ci_v], add=True)
            return 0
        lax.fori_loop(0, _NCHP, chunk, 0)

        plsc.subcore_barrier()

        @pl.when(s < _NDUMP)
        def _():
            pltpu.sync_copy(deg_sh.at[pl.ds(s * _ROWS_PT, _ROWS_PT)],
                            deg_o.at[c, pl.ds(s * _ROWS_PT, _ROWS_PT)])

    return k(colp, rowp, pos128, zrows)


def _sc_gather(colp, rowp, ah0, ah1, bh0, bh1):
    @functools.partial(
        pl.kernel,
        mesh=_sc_mesh(),
        out_type=[
            jax.ShapeDtypeStruct((2, E, 128), jnp.float32),
            jax.ShapeDtypeStruct((2, E, 128), jnp.float32),
        ],
        scratch_types=[
            pltpu.VMEM((_CH,), jnp.int32),
            pltpu.VMEM((_CH,), jnp.int32),
            pltpu.VMEM((_CH, 128), jnp.float32),
            pltpu.VMEM((_CH, 128), jnp.float32),
            pltpu.SemaphoreType.DMA,
            pltpu.SemaphoreType.DMA,
        ],
    )
    def k(col_h, row_h, a0_h, a1_h, b0_h, b1_h, ag_o, bg_o,
          ci_v, ri_v, a_v, b_v, sem1, sem2):
        c = lax.axis_index("c")
        s = lax.axis_index("s")
        base = s * _EPT

        def chunk(kk, _):
            off = base + kk * _CH
            pltpu.sync_copy(col_h.at[pl.ds(off, _CH)], ci_v)
            pltpu.sync_copy(row_h.at[pl.ds(off, _CH)], ri_v)

            @pl.when(c == 0)
            def _():
                cp1 = pltpu.async_copy(a0_h.at[ci_v], a_v, sem1)
                cp2 = pltpu.async_copy(b0_h.at[ri_v], b_v, sem2)
                cp1.wait()
                cp2.wait()

            @pl.when(c == 1)
            def _():
                cp1 = pltpu.async_copy(a1_h.at[ci_v], a_v, sem1)
                cp2 = pltpu.async_copy(b1_h.at[ri_v], b_v, sem2)
                cp1.wait()
                cp2.wait()

            pltpu.sync_copy(a_v, ag_o.at[c, pl.ds(off, _CH)])
            pltpu.sync_copy(b_v, bg_o.at[c, pl.ds(off, _CH)])
            return 0
        lax.fori_loop(0, _NCHUNK, chunk, 0)

    return k(colp, rowp, ah0, ah1, bh0, bh1)


def _sc_scatter(colp, p0, p1, zrows):
    @functools.partial(
        pl.kernel,
        mesh=_sc_mesh(),
        out_type=jax.ShapeDtypeStruct((2, N, 128), jnp.float32),
        scratch_types=[
            pltpu.VMEM((_CH,), jnp.int32),
            pltpu.VMEM((_CH, 128), jnp.float32),
            pltpu.VMEM_SHARED((N, 128), jnp.float32),
            pltpu.SemaphoreType.DMA,
        ],
    )
    def k(col_h, p0_h, p1_h, z_h, s_o, ci_v, p_v, s_sh, sem1):
        c = lax.axis_index("c")
        s = lax.axis_index("s")
        base = s * _EPT

        @pl.when(s < _NDUMP)
        def _():
            pltpu.sync_copy(z_h, s_sh.at[pl.ds(s * _ROWS_PT, _ROWS_PT)])
        plsc.subcore_barrier()

        def chunk(kk, _):
            off = base + kk * _CH
            pltpu.sync_copy(col_h.at[pl.ds(off, _CH)], ci_v)

            @pl.when(c == 0)
            def _():
                pltpu.async_copy(p0_h.at[pl.ds(off, _CH)], p_v, sem1).wait()

            @pl.when(c == 1)
            def _():
                pltpu.async_copy(p1_h.at[pl.ds(off, _CH)], p_v, sem1).wait()

            pltpu.sync_copy(p_v, s_sh.at[ci_v], add=True)
            return 0
        lax.fori_loop(0, _NCHUNK, chunk, 0)

        plsc.subcore_barrier()

        @pl.when(s < _NDUMP)
        def _():
            pltpu.sync_copy(s_sh.at[pl.ds(s * _ROWS_PT, _ROWS_PT)],
                            s_o.at[c, pl.ds(s * _ROWS_PT, _ROWS_PT)])

    return k(colp, p0, p1, zrows)


def _pre_body(x_ref, pos_ref,
              win, bin_, gin, bln,
              p1a, p1ab, p1b, p1bb, p2a, p2ab, p2b, p2bb,
              wa1, wb1,
              xe1_o, a0_o, a1_o, b0_o, b1_o, pe2_o):
    x = x_ref[...]
    pos = pos_ref[...]
    h = _silu(_ln(jnp.dot(x, win[...], preferred_element_type=jnp.float32)
                  + bin_[...], gin[...], bln[...]))
    pe1 = jnp.dot(_silu(jnp.dot(pos, p1a[...], preferred_element_type=jnp.float32)
                        + p1ab[...]), p1b[...],
                  preferred_element_type=jnp.float32) + p1bb[...]
    pe2 = jnp.dot(_silu(jnp.dot(pos, p2a[...], preferred_element_type=jnp.float32)
                        + p2ab[...]), p2b[...],
                  preferred_element_type=jnp.float32) + p2bb[...]
    xe1 = h + pe1
    xe1_o[...] = xe1
    pe2_o[...] = pe2
    a = jnp.dot(xe1, wa1[...], preferred_element_type=jnp.float32)
    b = jnp.dot(xe1, wb1[...], preferred_element_type=jnp.float32)
    a0_o[...] = a[:, :128]
    a1_o[...] = a[:, 128:]
    b0_o[...] = b[:, :128]
    b1_o[...] = b[:, 128:]


def _k_pre(x, pos8, W):
    grid = (N // _NB,)
    bspec_n = pl.BlockSpec((_NB, 256), lambda i: (i, 0))
    bspec_h = pl.BlockSpec((_NB, 128), lambda i: (i, 0))
    bspec_p = pl.BlockSpec((_NB, 8), lambda i: (i, 0))
    cw = lambda shape: pl.BlockSpec(shape, lambda i: tuple(0 for _ in shape))
    out_shapes = ([jax.ShapeDtypeStruct((N, 256), jnp.float32)]
                  + [jax.ShapeDtypeStruct((N, 128), jnp.float32)] * 4
                  + [jax.ShapeDtypeStruct((N, 256), jnp.float32)])
    return pl.pallas_call(
        _pre_body,
        grid=grid,
        in_specs=[bspec_n, bspec_p,
                  cw((256, 256)), cw((1, 256)), cw((1, 256)), cw((1, 256)),
                  cw((8, 128)), cw((1, 128)), cw((128, 256)), cw((1, 256)),
                  cw((8, 128)), cw((1, 128)), cw((128, 256)), cw((1, 256)),
                  cw((256, 256)), cw((256, 256))],
        out_specs=[bspec_n, bspec_h, bspec_h, bspec_h, bspec_h, bspec_n],
        out_shape=out_shapes,
    )(x, pos8, *W)


def _ea_body(ev_ref, ea_o):
    ev = ev_ref[...][:, :3]
    ed = jnp.sqrt(jnp.sum(ev * ev, axis=-1, keepdims=True))
    dirn = ev / (ed + 1e-8)
    dn = ed / (RADIUS + 1e-8)
    z = jnp.zeros_like(ev_ref[...][:, :4])
    ea_o[...] = jnp.concatenate([dirn, dn, z], axis=-1)


def _k_ea(evec):
    grid = (E // _EB,)
    bspec16 = pl.BlockSpec((_EB, 16), lambda i: (i, 0))
    bspec8 = pl.BlockSpec((_EB, 8), lambda i: (i, 0))
    return pl.pallas_call(
        _ea_body,
        grid=grid,
        in_specs=[bspec16],
        out_specs=bspec8,
        out_shape=jax.ShapeDtypeStruct((E, 8), jnp.float32),
    )(evec)


def _edge_body(a0, a1, b0, b1, ea, wc, c0, seg, gg, bb, p0_o, p1_o):
    G = jnp.concatenate([a0[0] + b0[0], a1[0] + b1[0]], axis=-1)
    G = G + jnp.dot(ea[...], wc[...], preferred_element_type=jnp.float32) + c0[...]
    segm = seg[...]
    dims = jnp.sum(segm, axis=1, keepdims=True).T
    mu = jnp.dot(G, segm.T, preferred_element_type=jnp.float32) / dims
    muf = jnp.dot(mu, segm, preferred_element_type=jnp.float32)
    cen = G - muf
    var = jnp.dot(cen * cen, segm.T, preferred_element_type=jnp.float32) / dims
    denom = jax.lax.rsqrt(jnp.dot(var, segm, preferred_element_type=jnp.float32) + 1e-5)
    P = _silu(gg[...] * cen * denom + bb[...])
    p0_o[...] = P[:, :128]
    p1_o[...] = P[:, 128:]


def _k_edge(Ag, Bg, ea8, wc, c0, seg, gcat, bcat):
    grid = (E // _EB,)
    h0 = pl.BlockSpec((1, _EB, 128), lambda i: (0, i, 0))
    h1 = pl.BlockSpec((1, _EB, 128), lambda i: (1, i, 0))
    be = pl.BlockSpec((_EB, 8), lambda i: (i, 0))
    bh = pl.BlockSpec((_EB, 128), lambda i: (i, 0))
    cw = lambda shape: pl.BlockSpec(shape, lambda i: tuple(0 for _ in shape))
    return pl.pallas_call(
        _edge_body,
        grid=grid,
        in_specs=[h0, h1, h0, h1, be,
                  cw((8, 256)), cw((1, 256)), cw((3, 256)), cw((1, 256)), cw((1, 256))],
        out_specs=[bh, bh],
        out_shape=[jax.ShapeDtypeStruct((E, 128), jnp.float32)] * 2,
    )(Ag, Ag, Bg, Bg, ea8, wc, c0, seg, gcat, bcat)


def _node_body(has_next, s0, s1, deg_ref, xe_ref, pen_ref,
               w2bd, b2, seg, ws1, bs1, ws2, bs2,
               wu1x, wu1w, bu1, gu, bu, wu2, bu2, gn, bn,
               wan, wbn, *outs):
    S = jnp.concatenate([s0[0], s1[0]], axis=-1)
    xe = xe_ref[...]
    agg = (jnp.dot(S, w2bd[...], preferred_element_type=jnp.float32)
           + deg_ref[...] * b2[...])
    t = _silu(jnp.dot(agg, ws1[...], preferred_element_type=jnp.float32) + bs1[...])
    logit = jnp.dot(t, ws2[...], preferred_element_type=jnp.float32) + bs2[...]
    aw = jax.nn.softmax(logit, axis=-1)
    weighted = agg * jnp.dot(aw, seg[...], preferred_element_type=jnp.float32)
    u = (jnp.dot(xe, wu1x[...], preferred_element_type=jnp.float32)
         + jnp.dot(weighted, wu1w[...], preferred_element_type=jnp.float32)
         + bu1[...])
    u = _silu(_ln(u, gu[...], bu[...]))
    u = jnp.dot(u, wu2[...], preferred_element_type=jnp.float32) + bu2[...]
    h = _ln(u + xe, gn[...], bn[...])
    if has_next:
        xe2_o, a0_o, a1_o, b0_o, b1_o = outs
        xe2 = h + pen_ref[...]
        xe2_o[...] = xe2
        a = jnp.dot(xe2, wan[...], preferred_element_type=jnp.float32)
        b = jnp.dot(xe2, wbn[...], preferred_element_type=jnp.float32)
        a0_o[...] = a[:, :128]
        a1_o[...] = a[:, 128:]
        b0_o[...] = b[:, :128]
        b1_o[...] = b[:, 128:]
    else:
        outs[0][...] = h


def _k_node(S2h, deg2d, xe, pe_next, W, has_next):
    grid = (N // _NB,)
    bspec_n = pl.BlockSpec((_NB, 256), lambda i: (i, 0))
    bspec_h = pl.BlockSpec((_NB, 128), lambda i: (i, 0))
    bspec_d = pl.BlockSpec((_NB, 1), lambda i: (i, 0))
    s0 = pl.BlockSpec((1, _NB, 128), lambda i: (0, i, 0))
    s1 = pl.BlockSpec((1, _NB, 128), lambda i: (1, i, 0))
    cw = lambda shape: pl.BlockSpec(shape, lambda i: tuple(0 for _ in shape))
    w_specs = [cw((256, 256)), cw((1, 256)), cw((3, 256)),
               cw((256, 64)), cw((1, 64)), cw((64, 3)), cw((1, 3)),
               cw((256, 512)), cw((256, 512)), cw((1, 512)), cw((1, 512)), cw((1, 512)),
               cw((512, 256)), cw((1, 256)), cw((1, 256)), cw((1, 256)),
               cw((256, 256)), cw((256, 256))]
    if has_next:
        out_specs = [bspec_n, bspec_h, bspec_h, bspec_h, bspec_h]
        out_shape = ([jax.ShapeDtypeStruct((N, 256), jnp.float32)]
                     + [jax.ShapeDtypeStruct((N, 128), jnp.float32)] * 4)
    else:
        out_specs = [bspec_n]
        out_shape = [jax.ShapeDtypeStruct((N, 256), jnp.float32)]
    return pl.pallas_call(
        functools.partial(_node_body, has_next),
        grid=grid,
        in_specs=[s0, s1, bspec_d, bspec_n, bspec_n] + w_specs,
        out_specs=out_specs,
        out_shape=out_shape,
    )(S2h, S2h, deg2d, xe, pe_next, *W)


def _layer_consts(lp):
    WAs, WBs, Wcs, c0s, gc, bc, b2c = [], [], [], [], [], [], []
    blocks = []
    for i, s in enumerate(SCALE_FACTORS):
        mp = lp['msg'][i]
        W1 = mp['l1']['w']
        WAs.append(W1[:256])
        WBs.append(W1[256:512])
        Wcs.append(W1[512:516])
        c0s.append(s * W1[516] + mp['l1']['b'])
        gc.append(mp['ln']['g'])
        bc.append(mp['ln']['b'])
        blocks.append(mp['l2']['w'])
        b2c.append(mp['l2']['b'])
    W2bd = jax.scipy.linalg.block_diag(*blocks)
    Wc = jnp.concatenate(Wcs, 1)
    Wc8 = jnp.pad(Wc, ((0, 4), (0, 0)))
    return dict(
        WA=jnp.concatenate(WAs, 1), WB=jnp.concatenate(WBs, 1),
        Wc8=Wc8, c0=jnp.concatenate(c0s),
        g=jnp.concatenate(gc), b=jnp.concatenate(bc),
        W2bd=W2bd, b2=jnp.concatenate(b2c))


def kernel(x, pos, edge_index, batch, params):
    seg = jnp.asarray(_SEG_NP)
    row = edge_index[0].astype(jnp.int32)
    col = edge_index[1].astype(jnp.int32)

    L1, L2 = params['layers'][0], params['layers'][1]
    C1, C2 = _layer_consts(L1), _layer_consts(L2)

    pos128 = jnp.pad(pos, ((0, 0), (0, 125)))
    zrows = jnp.zeros((_ROWS_PT, 128), jnp.float32)

    evec, degp = _sc_prologue(col, row, pos128, zrows)
    deg2d = (degp[0, :, 0] + degp[1, :, 0])[:, None]
    ea8 = _k_ea(evec)

    pos8 = jnp.pad(pos, ((0, 0), (0, 5)))
    ip = params['input_proj']
    pe_w = lambda lp, k: lp['pos_enc'][k]
    pre_W = [
        ip['lin']['w'], _row(ip['lin']['b']), _row(ip['ln']['g']), _row(ip['ln']['b']),
        jnp.pad(pe_w(L1, 'l1')['w'], ((0, 5), (0, 0))), _row(pe_w(L1, 'l1')['b']),
        pe_w(L1, 'l2')['w'], _row(pe_w(L1, 'l2')['b']),
        jnp.pad(pe_w(L2, 'l1')['w'], ((0, 5), (0, 0))), _row(pe_w(L2, 'l1')['b']),
        pe_w(L2, 'l2')['w'], _row(pe_w(L2, 'l2')['b']),
        C1['WA'], C1['WB'],
    ]
    xe1, a0, a1, b0, b1, pe2 = _k_pre(x, pos8, pre_W)

    def edge_phase(a0, a1, b0, b1, C):
        Ag, Bg = _sc_gather(col, row, a0, a1, b0, b1)
        P0, P1 = _k_edge(Ag, Bg, ea8, C['Wc8'], _row(C['c0']), seg,
                         _row(C['g']), _row(C['b']))
        return _sc_scatter(col, P0, P1, zrows)

    def node_W(lp, C, Cn):
        up = lp['update']
        sa = lp['scale_att']
        Wu1 = up['l1']['w']
        return [
            C['W2bd'], _row(C['b2']), seg,
            sa['l1']['w'], _row(sa['l1']['b']), sa['l2']['w'], _row(sa['l2']['b']),
            Wu1[:256], Wu1[256:], _row(up['l1']['b']),
            _row(up['ln']['g']), _row(up['ln']['b']),
            up['l2']['w'], _row(up['l2']['b']),
            _row(lp['norm']['g']), _row(lp['norm']['b']),
            (Cn['WA'] if Cn is not None else C['WA']),
            (Cn['WB'] if Cn is not None else C['WB']),
        ]

    S1 = edge_phase(a0, a1, b0, b1, C1)
    xe2, a0, a1, b0, b1 = _k_node(S1, deg2d, xe1, pe2, node_W(L1, C1, C2), True)
    S2 = edge_phase(a0, a1, b0, b1, C2)
    (h,) = _k_node(S2, deg2d, xe2, pe2, node_W(L2, C2, None), False)

    ones = jnp.ones((N,), jnp.float32)
    cnt = jax.ops.segment_sum(ones, batch, num_segments=NUM_GRAPHS)
    mean_pool = jax.ops.segment_sum(h, batch, num_segments=NUM_GRAPHS) / jnp.maximum(cnt, 1.0)[:, None]
    max_pool = jax.ops.segment_max(h, batch, num_segments=NUM_GRAPHS)
    ap = params['att_pool']
    att = jax.nn.softmax((_silu(h @ ap['l1']['w'] + ap['l1']['b'])) @ ap['l2']['w'] + ap['l2']['b'], axis=0)
    att_pool = jax.ops.segment_sum(h * att, batch, num_segments=NUM_GRAPHS)
    combined = jnp.concatenate([mean_pool, max_pool, att_pool], -1)
    pf = params['pool_fusion']
    pooled = _silu(_ln(combined @ pf['lin']['w'] + pf['lin']['b'], pf['ln']['g'], pf['ln']['b']))
    t = params['topo']

    def head(hp, x_):
        return (_silu(x_ @ hp['l1']['w'] + hp['l1']['b'])) @ hp['l2']['w'] + hp['l2']['b']

    chern = jnp.tanh(head(t['chern'], pooled))
    z2 = jax.nn.sigmoid(head(t['z2'], pooled))
    gap = jax.nn.softplus(head(t['gap'], pooled))
    sym = head(t['sym'], pooled)
    topo_feat = jnp.concatenate([chern, z2, gap, sym], -1)
    f = jnp.concatenate([pooled, topo_feat], -1)
    op = params['output_proj']
    f = _silu(_ln(f @ op['l1']['w'] + op['l1']['b'], op['ln1']['g'], op['ln1']['b']))
    f = _silu(_ln(f @ op['l2']['w'] + op['l2']['b'], op['ln2']['g'], op['ln2']['b']))
    out = f @ op['l3']['w'] + op['l3']['b']
    th = params['topo_head']
    logits = head(th, out)
    return out, logits, topo_feat


# speedup vs baseline: 3.8101x; 1.1131x over previous
import functools

import jax
import jax.numpy as jnp
import numpy as np
from jax import lax
from jax.experimental import pallas as pl
from jax.experimental.pallas import tpu as pltpu
from jax.experimental.pallas import tpu_sc as plsc

N = 10000
E = 160000
H = 256
NUM_GRAPHS = 16
RADIUS = 4.0
SCALE_FACTORS = (1.0, 2.0, 4.0)

_SEG_NP = np.zeros((3, 256), np.float32)
_SEG_NP[0, :86] = 1.0
_SEG_NP[1, 86:171] = 1.0
_SEG_NP[2, 171:256] = 1.0

_NB = 1000
_EB = 2000

_NSUB = 16
_CH = 80
_EPT = E // _NSUB
_NCHUNK = _EPT // _CH
_CHP = 40
_EPT32 = E // 32
_NCHP = _EPT32 // _CHP
_ROWS_PT = 1000
_NDUMP = N // _ROWS_PT

_sc_mesh = lambda: plsc.VectorSubcoreMesh(core_axis_name="c", subcore_axis_name="s")


def _silu(x):
    return x * jax.nn.sigmoid(x)


def _ln(x, g, b):
    mu = jnp.mean(x, axis=-1, keepdims=True)
    var = jnp.mean((x - mu) ** 2, axis=-1, keepdims=True)
    return g * (x - mu) * jax.lax.rsqrt(var + 1e-5) + b


def _row(v):
    return v.reshape(1, -1)


def _sc_prologue(colp, rowp, pos128, zrows):
    @functools.partial(
        pl.kernel,
        mesh=_sc_mesh(),
        out_type=[
            jax.ShapeDtypeStruct((E, 16), jnp.float32),
            jax.ShapeDtypeStruct((2, N, 128), jnp.float32),
        ],
        scratch_types=[
            pltpu.VMEM((_CHP,), jnp.int32),
            pltpu.VMEM((_CHP,), jnp.int32),
            pltpu.VMEM((_CHP, 128), jnp.float32),
            pltpu.VMEM((_CHP, 128), jnp.float32),
            pltpu.VMEM((_CHP, 128), jnp.float32),
            pltpu.VMEM((_CHP, 16), jnp.float32),
            pltpu.VMEM_SHARED((N, 128), jnp.float32),
            pltpu.SemaphoreType.DMA,
            pltpu.SemaphoreType.DMA,
        ],
    )
    def k(col_h, row_h, pos_h, z_h, evec_o, deg_o,
          ci_v, ri_v, pr_v, pc_v, ones_v, vec_v, deg_sh, sem1, sem2):
        c = lax.axis_index("c")
        s = lax.axis_index("s")
        wid = c * _NSUB + s
        base = wid * _EPT32

        def fill(i, _):
            for j in range(8):
                ones_v[i, pl.ds(j * 16, 16)] = jnp.full((16,), 1.0, jnp.float32)
            return 0
        lax.fori_loop(0, _CHP, fill, 0)

        @pl.when(s < _NDUMP)
        def _():
            pltpu.sync_copy(z_h, deg_sh.at[pl.ds(s * _ROWS_PT, _ROWS_PT)])
        plsc.subcore_barrier()

        def chunk(kk, _):
            off = base + kk * _CHP
            pltpu.sync_copy(col_h.at[pl.ds(off, _CHP)], ci_v)
            pltpu.sync_copy(row_h.at[pl.ds(off, _CHP)], ri_v)
            cp1 = pltpu.async_copy(pos_h.at[ri_v], pr_v, sem1)
            cp2 = pltpu.async_copy(pos_h.at[ci_v], pc_v, sem2)
            cp1.wait()
            cp2.wait()

            def sub(i, _):
                vec_v[i, pl.ds(0, 16)] = (pr_v[i, pl.ds(0, 16)]
                                          - pc_v[i, pl.ds(0, 16)])
                return 0
            lax.fori_loop(0, _CHP, sub, 0)
            pltpu.sync_copy(vec_v, evec_o.at[pl.ds(off, _CHP)])
            pltpu.sync_copy(ones_v, deg_sh.at[ci_v], add=True)
            return 0
        lax.fori_loop(0, _NCHP, chunk, 0)

        plsc.subcore_barrier()

        @pl.when(s < _NDUMP)
        def _():
            pltpu.sync_copy(deg_sh.at[pl.ds(s * _ROWS_PT, _ROWS_PT)],
                            deg_o.at[c, pl.ds(s * _ROWS_PT, _ROWS_PT)])

    return k(colp, rowp, pos128, zrows)


def _sc_gather(colp, rowp, ah0, ah1, bh0, bh1):
    @functools.partial(
        pl.kernel,
        mesh=_sc_mesh(),
        out_type=[
            jax.ShapeDtypeStruct((2, E, 128), jnp.float32),
            jax.ShapeDtypeStruct((2, E, 128), jnp.float32),
        ],
        scratch_types=[
            pltpu.VMEM((_CH,), jnp.int32),
            pltpu.VMEM((_CH,), jnp.int32),
            pltpu.VMEM((_CH, 128), jnp.float32),
            pltpu.VMEM((_CH, 128), jnp.float32),
            pltpu.SemaphoreType.DMA,
            pltpu.SemaphoreType.DMA,
        ],
    )
    def k(col_h, row_h, a0_h, a1_h, b0_h, b1_h, ag_o, bg_o,
          ci_v, ri_v, a_v, b_v, sem1, sem2):
        c = lax.axis_index("c")
        s = lax.axis_index("s")
        base = s * _EPT

        def chunk(kk, _):
            off = base + kk * _CH
            pltpu.sync_copy(col_h.at[pl.ds(off, _CH)], ci_v)
            pltpu.sync_copy(row_h.at[pl.ds(off, _CH)], ri_v)

            @pl.when(c == 0)
            def _():
                cp1 = pltpu.async_copy(a0_h.at[ci_v], a_v, sem1)
                cp2 = pltpu.async_copy(b0_h.at[ri_v], b_v, sem2)
                cp1.wait()
                cp2.wait()

            @pl.when(c == 1)
            def _():
                cp1 = pltpu.async_copy(a1_h.at[ci_v], a_v, sem1)
                cp2 = pltpu.async_copy(b1_h.at[ri_v], b_v, sem2)
                cp1.wait()
                cp2.wait()

            pltpu.sync_copy(a_v, ag_o.at[c, pl.ds(off, _CH)])
            pltpu.sync_copy(b_v, bg_o.at[c, pl.ds(off, _CH)])
            return 0
        lax.fori_loop(0, _NCHUNK, chunk, 0)

    return k(colp, rowp, ah0, ah1, bh0, bh1)


def _sc_scatter(colp, p0, p1, zrows):
    @functools.partial(
        pl.kernel,
        mesh=_sc_mesh(),
        out_type=jax.ShapeDtypeStruct((2, N, 128), jnp.float32),
        scratch_types=[
            pltpu.VMEM((_CH,), jnp.int32),
            pltpu.VMEM((_CH, 128), jnp.float32),
            pltpu.VMEM_SHARED((N, 128), jnp.float32),
            pltpu.SemaphoreType.DMA,
        ],
    )
    def k(col_h, p0_h, p1_h, z_h, s_o, ci_v, p_v, s_sh, sem1):
        c = lax.axis_index("c")
        s = lax.axis_index("s")
        base = s * _EPT

        @pl.when(s < _NDUMP)
        def _():
            pltpu.sync_copy(z_h, s_sh.at[pl.ds(s * _ROWS_PT, _ROWS_PT)])
        plsc.subcore_barrier()

        def chunk(kk, _):
            off = base + kk * _CH
            pltpu.sync_copy(col_h.at[pl.ds(off, _CH)], ci_v)

            @pl.when(c == 0)
            def _():
                pltpu.async_copy(p0_h.at[pl.ds(off, _CH)], p_v, sem1).wait()

            @pl.when(c == 1)
            def _():
                pltpu.async_copy(p1_h.at[pl.ds(off, _CH)], p_v, sem1).wait()

            pltpu.sync_copy(p_v, s_sh.at[ci_v], add=True)
            return 0
        lax.fori_loop(0, _NCHUNK, chunk, 0)

        plsc.subcore_barrier()

        @pl.when(s < _NDUMP)
        def _():
            pltpu.sync_copy(s_sh.at[pl.ds(s * _ROWS_PT, _ROWS_PT)],
                            s_o.at[c, pl.ds(s * _ROWS_PT, _ROWS_PT)])

    return k(colp, p0, p1, zrows)


def _pre_body(x_ref, pos_ref,
              win, bin_, gin, bln,
              p1a, p1ab, p1b, p1bb, p2a, p2ab, p2b, p2bb,
              wa1, wb1,
              xe1_o, a0_o, a1_o, b0_o, b1_o, pe2_o):
    x = x_ref[...]
    pos = pos_ref[...]
    h = _silu(_ln(jnp.dot(x, win[...], preferred_element_type=jnp.float32)
                  + bin_[...], gin[...], bln[...]))
    pe1 = jnp.dot(_silu(jnp.dot(pos, p1a[...], preferred_element_type=jnp.float32)
                        + p1ab[...]), p1b[...],
                  preferred_element_type=jnp.float32) + p1bb[...]
    pe2 = jnp.dot(_silu(jnp.dot(pos, p2a[...], preferred_element_type=jnp.float32)
                        + p2ab[...]), p2b[...],
                  preferred_element_type=jnp.float32) + p2bb[...]
    xe1 = h + pe1
    xe1_o[...] = xe1
    pe2_o[...] = pe2
    a = jnp.dot(xe1, wa1[...], preferred_element_type=jnp.float32)
    b = jnp.dot(xe1, wb1[...], preferred_element_type=jnp.float32)
    a0_o[...] = a[:, :128]
    a1_o[...] = a[:, 128:]
    b0_o[...] = b[:, :128]
    b1_o[...] = b[:, 128:]


def _k_pre(x, pos8, W):
    grid = (N // _NB,)
    bspec_n = pl.BlockSpec((_NB, 256), lambda i: (i, 0))
    bspec_h = pl.BlockSpec((_NB, 128), lambda i: (i, 0))
    bspec_p = pl.BlockSpec((_NB, 8), lambda i: (i, 0))
    cw = lambda shape: pl.BlockSpec(shape, lambda i: tuple(0 for _ in shape))
    out_shapes = ([jax.ShapeDtypeStruct((N, 256), jnp.float32)]
                  + [jax.ShapeDtypeStruct((N, 128), jnp.float32)] * 4
                  + [jax.ShapeDtypeStruct((N, 256), jnp.float32)])
    return pl.pallas_call(
        _pre_body,
        grid=grid,
        in_specs=[bspec_n, bspec_p,
                  cw((256, 256)), cw((1, 256)), cw((1, 256)), cw((1, 256)),
                  cw((8, 128)), cw((1, 128)), cw((128, 256)), cw((1, 256)),
                  cw((8, 128)), cw((1, 128)), cw((128, 256)), cw((1, 256)),
                  cw((256, 256)), cw((256, 256))],
        out_specs=[bspec_n, bspec_h, bspec_h, bspec_h, bspec_h, bspec_n],
        out_shape=out_shapes,
    )(x, pos8, *W)


def _ea_body(ev_ref, ea_o):
    ev = ev_ref[...][:, :3]
    ed = jnp.sqrt(jnp.sum(ev * ev, axis=-1, keepdims=True))
    dirn = ev / (ed + 1e-8)
    dn = ed / (RADIUS + 1e-8)
    z = jnp.zeros_like(ev_ref[...][:, :4])
    ea_o[...] = jnp.concatenate([dirn, dn, z], axis=-1)


def _k_ea(evec):
    grid = (E // _EB,)
    bspec16 = pl.BlockSpec((_EB, 16), lambda i: (i, 0))
    bspec8 = pl.BlockSpec((_EB, 8), lambda i: (i, 0))
    return pl.pallas_call(
        _ea_body,
        grid=grid,
        in_specs=[bspec16],
        out_specs=bspec8,
        out_shape=jax.ShapeDtypeStruct((E, 8), jnp.float32),
    )(evec)


def _edge_body(a0, a1, b0, b1, ea, wc, c0, seg, gg, bb, p0_o, p1_o):
    G = jnp.concatenate([a0[0] + b0[0], a1[0] + b1[0]], axis=-1)
    G = G + jnp.dot(ea[...], wc[...], preferred_element_type=jnp.float32) + c0[...]
    segm = seg[...]
    dims = jnp.sum(segm, axis=1, keepdims=True).T
    mu = jnp.dot(G, segm.T, preferred_element_type=jnp.float32) / dims
    muf = jnp.dot(mu, segm, preferred_element_type=jnp.float32)
    cen = G - muf
    var = jnp.dot(cen * cen, segm.T, preferred_element_type=jnp.float32) / dims
    denom = jax.lax.rsqrt(jnp.dot(var, segm, preferred_element_type=jnp.float32) + 1e-5)
    P = _silu(gg[...] * cen * denom + bb[...])
    p0_o[...] = P[:, :128]
    p1_o[...] = P[:, 128:]


def _k_edge(Ag, Bg, ea8, wc, c0, seg, gcat, bcat):
    grid = (E // _EB,)
    h0 = pl.BlockSpec((1, _EB, 128), lambda i: (0, i, 0))
    h1 = pl.BlockSpec((1, _EB, 128), lambda i: (1, i, 0))
    be = pl.BlockSpec((_EB, 8), lambda i: (i, 0))
    bh = pl.BlockSpec((_EB, 128), lambda i: (i, 0))
    cw = lambda shape: pl.BlockSpec(shape, lambda i: tuple(0 for _ in shape))
    return pl.pallas_call(
        _edge_body,
        grid=grid,
        in_specs=[h0, h1, h0, h1, be,
                  cw((8, 256)), cw((1, 256)), cw((3, 256)), cw((1, 256)), cw((1, 256))],
        out_specs=[bh, bh],
        out_shape=[jax.ShapeDtypeStruct((E, 128), jnp.float32)] * 2,
    )(Ag, Ag, Bg, Bg, ea8, wc, c0, seg, gcat, bcat)


def _node_body(has_next, s0, s1, deg_ref, xe_ref, pen_ref,
               w2bd, b2, seg, ws1, bs1, ws2, bs2,
               wu1x, wu1w, bu1, gu, bu, wu2, bu2, gn, bn,
               wan, wbn, *outs):
    S = jnp.concatenate([s0[0], s1[0]], axis=-1)
    xe = xe_ref[...]
    agg = (jnp.dot(S, w2bd[...], preferred_element_type=jnp.float32)
           + deg_ref[...] * b2[...])
    t = _silu(jnp.dot(agg, ws1[...], preferred_element_type=jnp.float32) + bs1[...])
    logit = jnp.dot(t, ws2[...], preferred_element_type=jnp.float32) + bs2[...]
    aw = jax.nn.softmax(logit, axis=-1)
    weighted = agg * jnp.dot(aw, seg[...], preferred_element_type=jnp.float32)
    u = (jnp.dot(xe, wu1x[...], preferred_element_type=jnp.float32)
         + jnp.dot(weighted, wu1w[...], preferred_element_type=jnp.float32)
         + bu1[...])
    u = _silu(_ln(u, gu[...], bu[...]))
    u = jnp.dot(u, wu2[...], preferred_element_type=jnp.float32) + bu2[...]
    h = _ln(u + xe, gn[...], bn[...])
    if has_next:
        xe2_o, a0_o, a1_o, b0_o, b1_o = outs
        xe2 = h + pen_ref[...]
        xe2_o[...] = xe2
        a = jnp.dot(xe2, wan[...], preferred_element_type=jnp.float32)
        b = jnp.dot(xe2, wbn[...], preferred_element_type=jnp.float32)
        a0_o[...] = a[:, :128]
        a1_o[...] = a[:, 128:]
        b0_o[...] = b[:, :128]
        b1_o[...] = b[:, 128:]
    else:
        outs[0][...] = h


def _k_node(S2h, deg2d, xe, pe_next, W, has_next):
    grid = (N // _NB,)
    bspec_n = pl.BlockSpec((_NB, 256), lambda i: (i, 0))
    bspec_h = pl.BlockSpec((_NB, 128), lambda i: (i, 0))
    bspec_d = pl.BlockSpec((_NB, 1), lambda i: (i, 0))
    s0 = pl.BlockSpec((1, _NB, 128), lambda i: (0, i, 0))
    s1 = pl.BlockSpec((1, _NB, 128), lambda i: (1, i, 0))
    cw = lambda shape: pl.BlockSpec(shape, lambda i: tuple(0 for _ in shape))
    w_specs = [cw((256, 256)), cw((1, 256)), cw((3, 256)),
               cw((256, 64)), cw((1, 64)), cw((64, 3)), cw((1, 3)),
               cw((256, 512)), cw((256, 512)), cw((1, 512)), cw((1, 512)), cw((1, 512)),
               cw((512, 256)), cw((1, 256)), cw((1, 256)), cw((1, 256)),
               cw((256, 256)), cw((256, 256))]
    if has_next:
        out_specs = [bspec_n, bspec_h, bspec_h, bspec_h, bspec_h]
        out_shape = ([jax.ShapeDtypeStruct((N, 256), jnp.float32)]
                     + [jax.ShapeDtypeStruct((N, 128), jnp.float32)] * 4)
    else:
        out_specs = [bspec_n]
        out_shape = [jax.ShapeDtypeStruct((N, 256), jnp.float32)]
    return pl.pallas_call(
        functools.partial(_node_body, has_next),
        grid=grid,
        in_specs=[s0, s1, bspec_d, bspec_n, bspec_n] + w_specs,
        out_specs=out_specs,
        out_shape=out_shape,
    )(S2h, S2h, deg2d, xe, pe_next, *W)


def _pool_body(h_ref, oh_ref,
               wa1, ba1, wa2, ba2,
               wpf, bpf, gpf, bbpf,
               wc1, bc1, wc2, bc2,
               wz1, bz1, wz2, bz2,
               wg1, bg1, wg2, bg2,
               ws1_, bs1_, ws2_, bs2_,
               wo1, bo1, go1, bo1n, wo2, bo2, go2, bo2n, wo3, bo3,
               wt1, bt1, wt2, bt2,
               out_o, log_o, topo_o,
               msum, mxp, attn, cnt, mse):
    i = pl.program_id(0)

    @pl.when(i == 0)
    def _():
        msum[...] = jnp.zeros_like(msum)
        mxp[...] = jnp.full_like(mxp, -3e38)
        attn[...] = jnp.zeros_like(attn)
        cnt[...] = jnp.zeros_like(cnt)
        mse[...] = jnp.concatenate(
            [jnp.full((1, 1), -3e38, jnp.float32),
             jnp.zeros((1, 1), jnp.float32)], axis=-1)

    h = h_ref[...]
    oh = oh_ref[...]
    ohT_dot = lambda rhs: jax.lax.dot_general(
        oh, rhs, (((0,), (0,)), ((), ())), preferred_element_type=jnp.float32)

    s = (jnp.dot(_silu(jnp.dot(h, wa1[...], preferred_element_type=jnp.float32)
                       + ba1[...]), wa2[...],
                 preferred_element_type=jnp.float32) + ba2[...])
    m_old = mse[0, 0]
    se_old = mse[0, 1]
    m_new = jnp.maximum(m_old, jnp.max(s))
    scale = jnp.exp(m_old - m_new)
    e = jnp.exp(s - m_new)
    se_new = se_old * scale + jnp.sum(e)
    mse[...] = jnp.concatenate([jnp.full((1, 1), m_new, jnp.float32),
                                jnp.full((1, 1), se_new, jnp.float32)], axis=-1)
    attn[...] = attn[...] * scale + ohT_dot(h * e)
    msum[...] = msum[...] + ohT_dot(h)
    cnt[...] = cnt[...] + ohT_dot(jnp.ones_like(s))
    rows = [jnp.max(jnp.where(oh[:, g:g + 1] > 0.0, h, -3e38), axis=0,
                    keepdims=True) for g in range(NUM_GRAPHS)]
    mxp[...] = jnp.maximum(mxp[...], jnp.concatenate(rows, axis=0))

    mean = msum[...] / jnp.maximum(cnt[...], 1.0)
    att_pool = attn[...] / mse[0, 1]
    combined = jnp.concatenate([mean, mxp[...], att_pool], axis=-1)
    pooled = _silu(_ln(jnp.dot(combined, wpf[...],
                               preferred_element_type=jnp.float32) + bpf[...],
                       gpf[...], bbpf[...]))

    def head(w1, b1, w2, b2):
        t = _silu(jnp.dot(pooled, w1[...], preferred_element_type=jnp.float32)
                  + b1[...])
        return jnp.dot(t, w2[...], preferred_element_type=jnp.float32) + b2[...]

    chern = jnp.tanh(head(wc1, bc1, wc2, bc2))
    z2 = jax.nn.sigmoid(head(wz1, bz1, wz2, bz2))
    gap = jax.nn.softplus(head(wg1, bg1, wg2, bg2))
    sym = head(ws1_, bs1_, ws2_, bs2_)
    topo = jnp.concatenate([chern, z2, gap, sym], axis=-1)
    f = jnp.concatenate([pooled, topo], axis=-1)
    f = _silu(_ln(jnp.dot(f, wo1[...], preferred_element_type=jnp.float32)
                  + bo1[...], go1[...], bo1n[...]))
    f = _silu(_ln(jnp.dot(f, wo2[...], preferred_element_type=jnp.float32)
                  + bo2[...], go2[...], bo2n[...]))
    out = jnp.dot(f, wo3[...], preferred_element_type=jnp.float32) + bo3[...]
    t2 = _silu(jnp.dot(out, wt1[...], preferred_element_type=jnp.float32)
               + bt1[...])
    logits = jnp.dot(t2, wt2[...], preferred_element_type=jnp.float32) + bt2[...]
    out_o[...] = out
    log_o[...] = logits
    topo_o[...] = topo


def _k_pool(h, onehot, params):
    ap = params['att_pool']
    pf = params['pool_fusion']
    t = params['topo']
    op = params['output_proj']
    th = params['topo_head']
    W = [
        ap['l1']['w'], _row(ap['l1']['b']), ap['l2']['w'], _row(ap['l2']['b']),
        pf['lin']['w'], _row(pf['lin']['b']), _row(pf['ln']['g']), _row(pf['ln']['b']),
        t['chern']['l1']['w'], _row(t['chern']['l1']['b']),
        t['chern']['l2']['w'], _row(t['chern']['l2']['b']),
        t['z2']['l1']['w'], _row(t['z2']['l1']['b']),
        t['z2']['l2']['w'], _row(t['z2']['l2']['b']),
        t['gap']['l1']['w'], _row(t['gap']['l1']['b']),
        t['gap']['l2']['w'], _row(t['gap']['l2']['b']),
        t['sym']['l1']['w'], _row(t['sym']['l1']['b']),
        t['sym']['l2']['w'], _row(t['sym']['l2']['b']),
        op['l1']['w'], _row(op['l1']['b']), _row(op['ln1']['g']), _row(op['ln1']['b']),
        op['l2']['w'], _row(op['l2']['b']), _row(op['ln2']['g']), _row(op['ln2']['b']),
        op['l3']['w'], _row(op['l3']['b']),
        th['l1']['w'], _row(th['l1']['b']), th['l2']['w'], _row(th['l2']['b']),
    ]
    grid = (N // _NB,)
    bspec_n = pl.BlockSpec((_NB, 256), lambda i: (i, 0))
    bspec_o = pl.BlockSpec((_NB, 16), lambda i: (i, 0))
    cw = lambda a: pl.BlockSpec(a.shape, lambda i: tuple(0 for _ in a.shape))
    return pl.pallas_call(
        _pool_body,
        grid=grid,
        in_specs=[bspec_n, bspec_o] + [cw(w) for w in W],
        out_specs=[pl.BlockSpec((16, 128), lambda i: (0, 0)),
                   pl.BlockSpec((16, 3), lambda i: (0, 0)),
                   pl.BlockSpec((16, 14), lambda i: (0, 0))],
        out_shape=[jax.ShapeDtypeStruct((16, 128), jnp.float32),
                   jax.ShapeDtypeStruct((16, 3), jnp.float32),
                   jax.ShapeDtypeStruct((16, 14), jnp.float32)],
        scratch_shapes=[pltpu.VMEM((16, 256), jnp.float32),
                        pltpu.VMEM((16, 256), jnp.float32),
                        pltpu.VMEM((16, 256), jnp.float32),
                        pltpu.VMEM((16, 1), jnp.float32),
                        pltpu.VMEM((1, 2), jnp.float32)],
    )(h, onehot, *W)


def _layer_consts(lp):
    WAs, WBs, Wcs, c0s, gc, bc, b2c = [], [], [], [], [], [], []
    blocks = []
    for i, s in enumerate(SCALE_FACTORS):
        mp = lp['msg'][i]
        W1 = mp['l1']['w']
        WAs.append(W1[:256])
        WBs.append(W1[256:512])
        Wcs.append(W1[512:516])
        c0s.append(s * W1[516] + mp['l1']['b'])
        gc.append(mp['ln']['g'])
        bc.append(mp['ln']['b'])
        blocks.append(mp['l2']['w'])
        b2c.append(mp['l2']['b'])
    W2bd = jax.scipy.linalg.block_diag(*blocks)
    Wc = jnp.concatenate(Wcs, 1)
    Wc8 = jnp.pad(Wc, ((0, 4), (0, 0)))
    return dict(
        WA=jnp.concatenate(WAs, 1), WB=jnp.concatenate(WBs, 1),
        Wc8=Wc8, c0=jnp.concatenate(c0s),
        g=jnp.concatenate(gc), b=jnp.concatenate(bc),
        W2bd=W2bd, b2=jnp.concatenate(b2c))


def kernel(x, pos, edge_index, batch, params):
    seg = jnp.asarray(_SEG_NP)
    row = edge_index[0].astype(jnp.int32)
    col = edge_index[1].astype(jnp.int32)

    L1, L2 = params['layers'][0], params['layers'][1]
    C1, C2 = _layer_consts(L1), _layer_consts(L2)

    pos128 = jnp.pad(pos, ((0, 0), (0, 125)))
    zrows = jnp.zeros((_ROWS_PT, 128), jnp.float32)

    evec, degp = _sc_prologue(col, row, pos128, zrows)
    deg2d = (degp[0, :, 0] + degp[1, :, 0])[:, None]
    ea8 = _k_ea(evec)

    pos8 = jnp.pad(pos, ((0, 0), (0, 5)))
    ip = params['input_proj']
    pe_w = lambda lp, k: lp['pos_enc'][k]
    pre_W = [
        ip['lin']['w'], _row(ip['lin']['b']), _row(ip['ln']['g']), _row(ip['ln']['b']),
        jnp.pad(pe_w(L1, 'l1')['w'], ((0, 5), (0, 0))), _row(pe_w(L1, 'l1')['b']),
        pe_w(L1, 'l2')['w'], _row(pe_w(L1, 'l2')['b']),
        jnp.pad(pe_w(L2, 'l1')['w'], ((0, 5), (0, 0))), _row(pe_w(L2, 'l1')['b']),
        pe_w(L2, 'l2')['w'], _row(pe_w(L2, 'l2')['b']),
        C1['WA'], C1['WB'],
    ]
    xe1, a0, a1, b0, b1, pe2 = _k_pre(x, pos8, pre_W)

    def edge_phase(a0, a1, b0, b1, C):
        Ag, Bg = _sc_gather(col, row, a0, a1, b0, b1)
        P0, P1 = _k_edge(Ag, Bg, ea8, C['Wc8'], _row(C['c0']), seg,
                         _row(C['g']), _row(C['b']))
        return _sc_scatter(col, P0, P1, zrows)

    def node_W(lp, C, Cn):
        up = lp['update']
        sa = lp['scale_att']
        Wu1 = up['l1']['w']
        return [
            C['W2bd'], _row(C['b2']), seg,
            sa['l1']['w'], _row(sa['l1']['b']), sa['l2']['w'], _row(sa['l2']['b']),
            Wu1[:256], Wu1[256:], _row(up['l1']['b']),
            _row(up['ln']['g']), _row(up['ln']['b']),
            up['l2']['w'], _row(up['l2']['b']),
            _row(lp['norm']['g']), _row(lp['norm']['b']),
            (Cn['WA'] if Cn is not None else C['WA']),
            (Cn['WB'] if Cn is not None else C['WB']),
        ]

    S1 = edge_phase(a0, a1, b0, b1, C1)
    xe2, a0, a1, b0, b1 = _k_node(S1, deg2d, xe1, pe2, node_W(L1, C1, C2), True)
    S2 = edge_phase(a0, a1, b0, b1, C2)
    (h,) = _k_node(S2, deg2d, xe2, pe2, node_W(L2, C2, None), False)

    onehot = (batch[:, None] == jnp.arange(NUM_GRAPHS)[None, :]).astype(jnp.float32)
    out, logits, topo_feat = _k_pool(h, onehot, params)
    return out, logits, topo_feat


# speedup vs baseline: 3.8474x; 1.0098x over previous
import functools

import jax
import jax.numpy as jnp
import numpy as np
from jax import lax
from jax.experimental import pallas as pl
from jax.experimental.pallas import tpu as pltpu
from jax.experimental.pallas import tpu_sc as plsc

N = 10000
E = 160000
H = 256
NUM_GRAPHS = 16
RADIUS = 4.0
SCALE_FACTORS = (1.0, 2.0, 4.0)

_SEG_NP = np.zeros((3, 256), np.float32)
_SEG_NP[0, :86] = 1.0
_SEG_NP[1, 86:171] = 1.0
_SEG_NP[2, 171:256] = 1.0

_NB = 1000
_EB = 2000

_NSUB = 16
_CH = 80
_EPT = E // _NSUB
_NCHUNK = _EPT // _CH
_CHP = 40
_EPT32 = E // 32
_NCHP = _EPT32 // _CHP
_ROWS_PT = 1000
_NDUMP = N // _ROWS_PT

_sc_mesh = lambda: plsc.VectorSubcoreMesh(core_axis_name="c", subcore_axis_name="s")


def _silu(x):
    return x * jax.nn.sigmoid(x)


def _ln(x, g, b):
    mu = jnp.mean(x, axis=-1, keepdims=True)
    var = jnp.mean((x - mu) ** 2, axis=-1, keepdims=True)
    return g * (x - mu) * jax.lax.rsqrt(var + 1e-5) + b


def _row(v):
    return v.reshape(1, -1)


def _sc_prologue(colp, rowp, pos128, zrows):
    @functools.partial(
        pl.kernel,
        mesh=_sc_mesh(),
        out_type=[
            jax.ShapeDtypeStruct((E, 16), jnp.float32),
            jax.ShapeDtypeStruct((2, N, 128), jnp.float32),
        ],
        scratch_types=[
            pltpu.VMEM((_CHP,), jnp.int32),
            pltpu.VMEM((_CHP,), jnp.int32),
            pltpu.VMEM((_CHP, 128), jnp.float32),
            pltpu.VMEM((_CHP, 128), jnp.float32),
            pltpu.VMEM((_CHP, 128), jnp.float32),
            pltpu.VMEM((_CHP, 16), jnp.float32),
            pltpu.VMEM_SHARED((N, 128), jnp.float32),
            pltpu.SemaphoreType.DMA,
            pltpu.SemaphoreType.DMA,
        ],
    )
    def k(col_h, row_h, pos_h, z_h, evec_o, deg_o,
          ci_v, ri_v, pr_v, pc_v, ones_v, vec_v, deg_sh, sem1, sem2):
        c = lax.axis_index("c")
        s = lax.axis_index("s")
        wid = c * _NSUB + s
        base = wid * _EPT32

        def fill(i, _):
            for j in range(8):
                ones_v[i, pl.ds(j * 16, 16)] = jnp.full((16,), 1.0, jnp.float32)
            return 0
        lax.fori_loop(0, _CHP, fill, 0)

        @pl.when(s < _NDUMP)
        def _():
            pltpu.sync_copy(z_h, deg_sh.at[pl.ds(s * _ROWS_PT, _ROWS_PT)])
        plsc.subcore_barrier()

        def chunk(kk, _):
            off = base + kk * _CHP
            pltpu.sync_copy(col_h.at[pl.ds(off, _CHP)], ci_v)
            pltpu.sync_copy(row_h.at[pl.ds(off, _CHP)], ri_v)
            cp1 = pltpu.async_copy(pos_h.at[ri_v], pr_v, sem1)
            cp2 = pltpu.async_copy(pos_h.at[ci_v], pc_v, sem2)
            cp1.wait()
            cp2.wait()

            def sub(i, _):
                vec_v[i, pl.ds(0, 16)] = (pr_v[i, pl.ds(0, 16)]
                                          - pc_v[i, pl.ds(0, 16)])
                return 0
            lax.fori_loop(0, _CHP, sub, 0)
            pltpu.sync_copy(vec_v, evec_o.at[pl.ds(off, _CHP)])
            pltpu.sync_copy(ones_v, deg_sh.at[ci_v], add=True)
            return 0
        lax.fori_loop(0, _NCHP, chunk, 0)

        plsc.subcore_barrier()

        @pl.when(s < _NDUMP)
        def _():
            pltpu.sync_copy(deg_sh.at[pl.ds(s * _ROWS_PT, _ROWS_PT)],
                            deg_o.at[c, pl.ds(s * _ROWS_PT, _ROWS_PT)])

    return k(colp, rowp, pos128, zrows)


def _sc_gather(colp, rowp, ah0, ah1, bh0, bh1):
    @functools.partial(
        pl.kernel,
        mesh=_sc_mesh(),
        out_type=jax.ShapeDtypeStruct((2, E, 128), jnp.float32),
        scratch_types=[
            pltpu.VMEM((_CH,), jnp.int32),
            pltpu.VMEM((_CH,), jnp.int32),
            pltpu.VMEM((_CH, 128), jnp.float32),
            pltpu.VMEM((_CH, 128), jnp.float32),
            pltpu.SemaphoreType.DMA,
            pltpu.SemaphoreType.DMA,
        ],
    )
    def k(col_h, row_h, a0_h, a1_h, b0_h, b1_h, g_o,
          ci_v, ri_v, a_v, b_v, sem1, sem2):
        c = lax.axis_index("c")
        s = lax.axis_index("s")
        base = s * _EPT

        def chunk(kk, _):
            off = base + kk * _CH
            pltpu.sync_copy(col_h.at[pl.ds(off, _CH)], ci_v)
            pltpu.sync_copy(row_h.at[pl.ds(off, _CH)], ri_v)

            @pl.when(c == 0)
            def _():
                cp1 = pltpu.async_copy(a0_h.at[ci_v], a_v, sem1)
                cp2 = pltpu.async_copy(b0_h.at[ri_v], b_v, sem2)
                cp1.wait()
                cp2.wait()

            @pl.when(c == 1)
            def _():
                cp1 = pltpu.async_copy(a1_h.at[ci_v], a_v, sem1)
                cp2 = pltpu.async_copy(b1_h.at[ri_v], b_v, sem2)
                cp1.wait()
                cp2.wait()

            def add(i, _):
                for j in range(8):
                    a_v[i, pl.ds(j * 16, 16)] = (a_v[i, pl.ds(j * 16, 16)]
                                                 + b_v[i, pl.ds(j * 16, 16)])
                return 0
            lax.fori_loop(0, _CH, add, 0)
            pltpu.sync_copy(a_v, g_o.at[c, pl.ds(off, _CH)])
            return 0
        lax.fori_loop(0, _NCHUNK, chunk, 0)

    return k(colp, rowp, ah0, ah1, bh0, bh1)


def _sc_scatter(colp, p0, p1, zrows):
    @functools.partial(
        pl.kernel,
        mesh=_sc_mesh(),
        out_type=jax.ShapeDtypeStruct((2, N, 128), jnp.float32),
        scratch_types=[
            pltpu.VMEM((_CH,), jnp.int32),
            pltpu.VMEM((_CH, 128), jnp.float32),
            pltpu.VMEM_SHARED((N, 128), jnp.float32),
            pltpu.SemaphoreType.DMA,
        ],
    )
    def k(col_h, p0_h, p1_h, z_h, s_o, ci_v, p_v, s_sh, sem1):
        c = lax.axis_index("c")
        s = lax.axis_index("s")
        base = s * _EPT

        @pl.when(s < _NDUMP)
        def _():
            pltpu.sync_copy(z_h, s_sh.at[pl.ds(s * _ROWS_PT, _ROWS_PT)])
        plsc.subcore_barrier()

        def chunk(kk, _):
            off = base + kk * _CH
            pltpu.sync_copy(col_h.at[pl.ds(off, _CH)], ci_v)

            @pl.when(c == 0)
            def _():
                pltpu.async_copy(p0_h.at[pl.ds(off, _CH)], p_v, sem1).wait()

            @pl.when(c == 1)
            def _():
                pltpu.async_copy(p1_h.at[pl.ds(off, _CH)], p_v, sem1).wait()

            pltpu.sync_copy(p_v, s_sh.at[ci_v], add=True)
            return 0
        lax.fori_loop(0, _NCHUNK, chunk, 0)

        plsc.subcore_barrier()

        @pl.when(s < _NDUMP)
        def _():
            pltpu.sync_copy(s_sh.at[pl.ds(s * _ROWS_PT, _ROWS_PT)],
                            s_o.at[c, pl.ds(s * _ROWS_PT, _ROWS_PT)])

    return k(colp, p0, p1, zrows)


def _pre_body(x_ref, pos_ref,
              win, bin_, gin, bln,
              p1a, p1ab, p1b, p1bb, p2a, p2ab, p2b, p2bb,
              wa1, wb1,
              xe1_o, a0_o, a1_o, b0_o, b1_o, pe2_o):
    x = x_ref[...]
    pos = pos_ref[...]
    h = _silu(_ln(jnp.dot(x, win[...], preferred_element_type=jnp.float32)
                  + bin_[...], gin[...], bln[...]))
    pe1 = jnp.dot(_silu(jnp.dot(pos, p1a[...], preferred_element_type=jnp.float32)
                        + p1ab[...]), p1b[...],
                  preferred_element_type=jnp.float32) + p1bb[...]
    pe2 = jnp.dot(_silu(jnp.dot(pos, p2a[...], preferred_element_type=jnp.float32)
                        + p2ab[...]), p2b[...],
                  preferred_element_type=jnp.float32) + p2bb[...]
    xe1 = h + pe1
    xe1_o[...] = xe1
    pe2_o[...] = pe2
    a = jnp.dot(xe1, wa1[...], preferred_element_type=jnp.float32)
    b = jnp.dot(xe1, wb1[...], preferred_element_type=jnp.float32)
    a0_o[...] = a[:, :128]
    a1_o[...] = a[:, 128:]
    b0_o[...] = b[:, :128]
    b1_o[...] = b[:, 128:]


def _k_pre(x, pos8, W):
    grid = (N // _NB,)
    bspec_n = pl.BlockSpec((_NB, 256), lambda i: (i, 0))
    bspec_h = pl.BlockSpec((_NB, 128), lambda i: (i, 0))
    bspec_p = pl.BlockSpec((_NB, 8), lambda i: (i, 0))
    cw = lambda shape: pl.BlockSpec(shape, lambda i: tuple(0 for _ in shape))
    out_shapes = ([jax.ShapeDtypeStruct((N, 256), jnp.float32)]
                  + [jax.ShapeDtypeStruct((N, 128), jnp.float32)] * 4
                  + [jax.ShapeDtypeStruct((N, 256), jnp.float32)])
    return pl.pallas_call(
        _pre_body,
        grid=grid,
        in_specs=[bspec_n, bspec_p,
                  cw((256, 256)), cw((1, 256)), cw((1, 256)), cw((1, 256)),
                  cw((8, 128)), cw((1, 128)), cw((128, 256)), cw((1, 256)),
                  cw((8, 128)), cw((1, 128)), cw((128, 256)), cw((1, 256)),
                  cw((256, 256)), cw((256, 256))],
        out_specs=[bspec_n, bspec_h, bspec_h, bspec_h, bspec_h, bspec_n],
        out_shape=out_shapes,
    )(x, pos8, *W)


def _ea_body(ev_ref, ea_o):
    ev = ev_ref[...][:, :3]
    ed = jnp.sqrt(jnp.sum(ev * ev, axis=-1, keepdims=True))
    dirn = ev / (ed + 1e-8)
    dn = ed / (RADIUS + 1e-8)
    z = jnp.zeros_like(ev_ref[...][:, :4])
    ea_o[...] = jnp.concatenate([dirn, dn, z], axis=-1)


def _k_ea(evec):
    grid = (E // _EB,)
    bspec16 = pl.BlockSpec((_EB, 16), lambda i: (i, 0))
    bspec8 = pl.BlockSpec((_EB, 8), lambda i: (i, 0))
    return pl.pallas_call(
        _ea_body,
        grid=grid,
        in_specs=[bspec16],
        out_specs=bspec8,
        out_shape=jax.ShapeDtypeStruct((E, 8), jnp.float32),
    )(evec)


def _edge_body(g0, g1, ea, wc, c0, seg, gg, bb, p0_o, p1_o):
    G = jnp.concatenate([g0[0], g1[0]], axis=-1)
    G = G + jnp.dot(ea[...], wc[...], preferred_element_type=jnp.float32) + c0[...]
    segm = seg[...]
    dims = jnp.sum(segm, axis=1, keepdims=True).T
    mu = jnp.dot(G, segm.T, preferred_element_type=jnp.float32) / dims
    muf = jnp.dot(mu, segm, preferred_element_type=jnp.float32)
    cen = G - muf
    var = jnp.dot(cen * cen, segm.T, preferred_element_type=jnp.float32) / dims
    denom = jax.lax.rsqrt(jnp.dot(var, segm, preferred_element_type=jnp.float32) + 1e-5)
    P = _silu(gg[...] * cen * denom + bb[...])
    p0_o[...] = P[:, :128]
    p1_o[...] = P[:, 128:]


def _k_edge(G, ea8, wc, c0, seg, gcat, bcat):
    grid = (E // _EB,)
    h0 = pl.BlockSpec((1, _EB, 128), lambda i: (0, i, 0))
    h1 = pl.BlockSpec((1, _EB, 128), lambda i: (1, i, 0))
    be = pl.BlockSpec((_EB, 8), lambda i: (i, 0))
    bh = pl.BlockSpec((_EB, 128), lambda i: (i, 0))
    cw = lambda shape: pl.BlockSpec(shape, lambda i: tuple(0 for _ in shape))
    return pl.pallas_call(
        _edge_body,
        grid=grid,
        in_specs=[h0, h1, be,
                  cw((8, 256)), cw((1, 256)), cw((3, 256)), cw((1, 256)), cw((1, 256))],
        out_specs=[bh, bh],
        out_shape=[jax.ShapeDtypeStruct((E, 128), jnp.float32)] * 2,
    )(G, G, ea8, wc, c0, seg, gcat, bcat)


def _node_body(has_next, s0, s1, deg_ref, xe_ref, pen_ref,
               w2bd, b2, seg, ws1, bs1, ws2, bs2,
               wu1x, wu1w, bu1, gu, bu, wu2, bu2, gn, bn,
               wan, wbn, *outs):
    S = jnp.concatenate([s0[0], s1[0]], axis=-1)
    xe = xe_ref[...]
    agg = (jnp.dot(S, w2bd[...], preferred_element_type=jnp.float32)
           + deg_ref[...] * b2[...])
    t = _silu(jnp.dot(agg, ws1[...], preferred_element_type=jnp.float32) + bs1[...])
    logit = jnp.dot(t, ws2[...], preferred_element_type=jnp.float32) + bs2[...]
    aw = jax.nn.softmax(logit, axis=-1)
    weighted = agg * jnp.dot(aw, seg[...], preferred_element_type=jnp.float32)
    u = (jnp.dot(xe, wu1x[...], preferred_element_type=jnp.float32)
         + jnp.dot(weighted, wu1w[...], preferred_element_type=jnp.float32)
         + bu1[...])
    u = _silu(_ln(u, gu[...], bu[...]))
    u = jnp.dot(u, wu2[...], preferred_element_type=jnp.float32) + bu2[...]
    h = _ln(u + xe, gn[...], bn[...])
    if has_next:
        xe2_o, a0_o, a1_o, b0_o, b1_o = outs
        xe2 = h + pen_ref[...]
        xe2_o[...] = xe2
        a = jnp.dot(xe2, wan[...], preferred_element_type=jnp.float32)
        b = jnp.dot(xe2, wbn[...], preferred_element_type=jnp.float32)
        a0_o[...] = a[:, :128]
        a1_o[...] = a[:, 128:]
        b0_o[...] = b[:, :128]
        b1_o[...] = b[:, 128:]
    else:
        outs[0][...] = h


def _k_node(S2h, deg2d, xe, pe_next, W, has_next):
    grid = (N // _NB,)
    bspec_n = pl.BlockSpec((_NB, 256), lambda i: (i, 0))
    bspec_h = pl.BlockSpec((_NB, 128), lambda i: (i, 0))
    bspec_d = pl.BlockSpec((_NB, 1), lambda i: (i, 0))
    s0 = pl.BlockSpec((1, _NB, 128), lambda i: (0, i, 0))
    s1 = pl.BlockSpec((1, _NB, 128), lambda i: (1, i, 0))
    cw = lambda shape: pl.BlockSpec(shape, lambda i: tuple(0 for _ in shape))
    w_specs = [cw((256, 256)), cw((1, 256)), cw((3, 256)),
               cw((256, 64)), cw((1, 64)), cw((64, 3)), cw((1, 3)),
               cw((256, 512)), cw((256, 512)), cw((1, 512)), cw((1, 512)), cw((1, 512)),
               cw((512, 256)), cw((1, 256)), cw((1, 256)), cw((1, 256)),
               cw((256, 256)), cw((256, 256))]
    if has_next:
        out_specs = [bspec_n, bspec_h, bspec_h, bspec_h, bspec_h]
        out_shape = ([jax.ShapeDtypeStruct((N, 256), jnp.float32)]
                     + [jax.ShapeDtypeStruct((N, 128), jnp.float32)] * 4)
    else:
        out_specs = [bspec_n]
        out_shape = [jax.ShapeDtypeStruct((N, 256), jnp.float32)]
    return pl.pallas_call(
        functools.partial(_node_body, has_next),
        grid=grid,
        in_specs=[s0, s1, bspec_d, bspec_n, bspec_n] + w_specs,
        out_specs=out_specs,
        out_shape=out_shape,
    )(S2h, S2h, deg2d, xe, pe_next, *W)


def _pool_body(h_ref, oh_ref,
               wa1, ba1, wa2, ba2,
               wpf, bpf, gpf, bbpf,
               wc1, bc1, wc2, bc2,
               wz1, bz1, wz2, bz2,
               wg1, bg1, wg2, bg2,
               ws1_, bs1_, ws2_, bs2_,
               wo1, bo1, go1, bo1n, wo2, bo2, go2, bo2n, wo3, bo3,
               wt1, bt1, wt2, bt2,
               out_o, log_o, topo_o,
               msum, mxp, attn, cnt, mse):
    i = pl.program_id(0)

    @pl.when(i == 0)
    def _():
        msum[...] = jnp.zeros_like(msum)
        mxp[...] = jnp.full_like(mxp, -3e38)
        attn[...] = jnp.zeros_like(attn)
        cnt[...] = jnp.zeros_like(cnt)
        mse[...] = jnp.concatenate(
            [jnp.full((1, 1), -3e38, jnp.float32),
             jnp.zeros((1, 1), jnp.float32)], axis=-1)

    h = h_ref[...]
    oh = oh_ref[...]
    ohT_dot = lambda rhs: jax.lax.dot_general(
        oh, rhs, (((0,), (0,)), ((), ())), preferred_element_type=jnp.float32)

    s = (jnp.dot(_silu(jnp.dot(h, wa1[...], preferred_element_type=jnp.float32)
                       + ba1[...]), wa2[...],
                 preferred_element_type=jnp.float32) + ba2[...])
    m_old = mse[0, 0]
    se_old = mse[0, 1]
    m_new = jnp.maximum(m_old, jnp.max(s))
    scale = jnp.exp(m_old - m_new)
    e = jnp.exp(s - m_new)
    se_new = se_old * scale + jnp.sum(e)
    mse[...] = jnp.concatenate([jnp.full((1, 1), m_new, jnp.float32),
                                jnp.full((1, 1), se_new, jnp.float32)], axis=-1)
    attn[...] = attn[...] * scale + ohT_dot(h * e)
    msum[...] = msum[...] + ohT_dot(h)
    cnt[...] = cnt[...] + ohT_dot(jnp.ones_like(s))
    rows = [jnp.max(jnp.where(oh[:, g:g + 1] > 0.0, h, -3e38), axis=0,
                    keepdims=True) for g in range(NUM_GRAPHS)]
    mxp[...] = jnp.maximum(mxp[...], jnp.concatenate(rows, axis=0))

    mean = msum[...] / jnp.maximum(cnt[...], 1.0)
    att_pool = attn[...] / mse[0, 1]
    combined = jnp.concatenate([mean, mxp[...], att_pool], axis=-1)
    pooled = _silu(_ln(jnp.dot(combined, wpf[...],
                               preferred_element_type=jnp.float32) + bpf[...],
                       gpf[...], bbpf[...]))

    def head(w1, b1, w2, b2):
        t = _silu(jnp.dot(pooled, w1[...], preferred_element_type=jnp.float32)
                  + b1[...])
        return jnp.dot(t, w2[...], preferred_element_type=jnp.float32) + b2[...]

    chern = jnp.tanh(head(wc1, bc1, wc2, bc2))
    z2 = jax.nn.sigmoid(head(wz1, bz1, wz2, bz2))
    gap = jax.nn.softplus(head(wg1, bg1, wg2, bg2))
    sym = head(ws1_, bs1_, ws2_, bs2_)
    topo = jnp.concatenate([chern, z2, gap, sym], axis=-1)
    f = jnp.concatenate([pooled, topo], axis=-1)
    f = _silu(_ln(jnp.dot(f, wo1[...], preferred_element_type=jnp.float32)
                  + bo1[...], go1[...], bo1n[...]))
    f = _silu(_ln(jnp.dot(f, wo2[...], preferred_element_type=jnp.float32)
                  + bo2[...], go2[...], bo2n[...]))
    out = jnp.dot(f, wo3[...], preferred_element_type=jnp.float32) + bo3[...]
    t2 = _silu(jnp.dot(out, wt1[...], preferred_element_type=jnp.float32)
               + bt1[...])
    logits = jnp.dot(t2, wt2[...], preferred_element_type=jnp.float32) + bt2[...]
    out_o[...] = out
    log_o[...] = logits
    topo_o[...] = topo


def _k_pool(h, onehot, params):
    ap = params['att_pool']
    pf = params['pool_fusion']
    t = params['topo']
    op = params['output_proj']
    th = params['topo_head']
    W = [
        ap['l1']['w'], _row(ap['l1']['b']), ap['l2']['w'], _row(ap['l2']['b']),
        pf['lin']['w'], _row(pf['lin']['b']), _row(pf['ln']['g']), _row(pf['ln']['b']),
        t['chern']['l1']['w'], _row(t['chern']['l1']['b']),
        t['chern']['l2']['w'], _row(t['chern']['l2']['b']),
        t['z2']['l1']['w'], _row(t['z2']['l1']['b']),
        t['z2']['l2']['w'], _row(t['z2']['l2']['b']),
        t['gap']['l1']['w'], _row(t['gap']['l1']['b']),
        t['gap']['l2']['w'], _row(t['gap']['l2']['b']),
        t['sym']['l1']['w'], _row(t['sym']['l1']['b']),
        t['sym']['l2']['w'], _row(t['sym']['l2']['b']),
        op['l1']['w'], _row(op['l1']['b']), _row(op['ln1']['g']), _row(op['ln1']['b']),
        op['l2']['w'], _row(op['l2']['b']), _row(op['ln2']['g']), _row(op['ln2']['b']),
        op['l3']['w'], _row(op['l3']['b']),
        th['l1']['w'], _row(th['l1']['b']), th['l2']['w'], _row(th['l2']['b']),
    ]
    grid = (N // _NB,)
    bspec_n = pl.BlockSpec((_NB, 256), lambda i: (i, 0))
    bspec_o = pl.BlockSpec((_NB, 16), lambda i: (i, 0))
    cw = lambda a: pl.BlockSpec(a.shape, lambda i: tuple(0 for _ in a.shape))
    return pl.pallas_call(
        _pool_body,
        grid=grid,
        in_specs=[bspec_n, bspec_o] + [cw(w) for w in W],
        out_specs=[pl.BlockSpec((16, 128), lambda i: (0, 0)),
                   pl.BlockSpec((16, 3), lambda i: (0, 0)),
                   pl.BlockSpec((16, 14), lambda i: (0, 0))],
        out_shape=[jax.ShapeDtypeStruct((16, 128), jnp.float32),
                   jax.ShapeDtypeStruct((16, 3), jnp.float32),
                   jax.ShapeDtypeStruct((16, 14), jnp.float32)],
        scratch_shapes=[pltpu.VMEM((16, 256), jnp.float32),
                        pltpu.VMEM((16, 256), jnp.float32),
                        pltpu.VMEM((16, 256), jnp.float32),
                        pltpu.VMEM((16, 1), jnp.float32),
                        pltpu.VMEM((1, 2), jnp.float32)],
    )(h, onehot, *W)


def _layer_consts(lp):
    WAs, WBs, Wcs, c0s, gc, bc, b2c = [], [], [], [], [], [], []
    blocks = []
    for i, s in enumerate(SCALE_FACTORS):
        mp = lp['msg'][i]
        W1 = mp['l1']['w']
        WAs.append(W1[:256])
        WBs.append(W1[256:512])
        Wcs.append(W1[512:516])
        c0s.append(s * W1[516] + mp['l1']['b'])
        gc.append(mp['ln']['g'])
        bc.append(mp['ln']['b'])
        blocks.append(mp['l2']['w'])
        b2c.append(mp['l2']['b'])
    W2bd = jax.scipy.linalg.block_diag(*blocks)
    Wc = jnp.concatenate(Wcs, 1)
    Wc8 = jnp.pad(Wc, ((0, 4), (0, 0)))
    return dict(
        WA=jnp.concatenate(WAs, 1), WB=jnp.concatenate(WBs, 1),
        Wc8=Wc8, c0=jnp.concatenate(c0s),
        g=jnp.concatenate(gc), b=jnp.concatenate(bc),
        W2bd=W2bd, b2=jnp.concatenate(b2c))


def kernel(x, pos, edge_index, batch, params):
    seg = jnp.asarray(_SEG_NP)
    row = edge_index[0].astype(jnp.int32)
    col = edge_index[1].astype(jnp.int32)

    L1, L2 = params['layers'][0], params['layers'][1]
    C1, C2 = _layer_consts(L1), _layer_consts(L2)

    pos128 = jnp.pad(pos, ((0, 0), (0, 125)))
    zrows = jnp.zeros((_ROWS_PT, 128), jnp.float32)

    evec, degp = _sc_prologue(col, row, pos128, zrows)
    deg2d = (degp[0, :, 0] + degp[1, :, 0])[:, None]
    ea8 = _k_ea(evec)

    pos8 = jnp.pad(pos, ((0, 0), (0, 5)))
    ip = params['input_proj']
    pe_w = lambda lp, k: lp['pos_enc'][k]
    pre_W = [
        ip['lin']['w'], _row(ip['lin']['b']), _row(ip['ln']['g']), _row(ip['ln']['b']),
        jnp.pad(pe_w(L1, 'l1')['w'], ((0, 5), (0, 0))), _row(pe_w(L1, 'l1')['b']),
        pe_w(L1, 'l2')['w'], _row(pe_w(L1, 'l2')['b']),
        jnp.pad(pe_w(L2, 'l1')['w'], ((0, 5), (0, 0))), _row(pe_w(L2, 'l1')['b']),
        pe_w(L2, 'l2')['w'], _row(pe_w(L2, 'l2')['b']),
        C1['WA'], C1['WB'],
    ]
    xe1, a0, a1, b0, b1, pe2 = _k_pre(x, pos8, pre_W)

    def edge_phase(a0, a1, b0, b1, C):
        G = _sc_gather(col, row, a0, a1, b0, b1)
        P0, P1 = _k_edge(G, ea8, C['Wc8'], _row(C['c0']), seg,
                         _row(C['g']), _row(C['b']))
        return _sc_scatter(col, P0, P1, zrows)

    def node_W(lp, C, Cn):
        up = lp['update']
        sa = lp['scale_att']
        Wu1 = up['l1']['w']
        return [
            C['W2bd'], _row(C['b2']), seg,
            sa['l1']['w'], _row(sa['l1']['b']), sa['l2']['w'], _row(sa['l2']['b']),
            Wu1[:256], Wu1[256:], _row(up['l1']['b']),
            _row(up['ln']['g']), _row(up['ln']['b']),
            up['l2']['w'], _row(up['l2']['b']),
            _row(lp['norm']['g']), _row(lp['norm']['b']),
            (Cn['WA'] if Cn is not None else C['WA']),
            (Cn['WB'] if Cn is not None else C['WB']),
        ]

    S1 = edge_phase(a0, a1, b0, b1, C1)
    xe2, a0, a1, b0, b1 = _k_node(S1, deg2d, xe1, pe2, node_W(L1, C1, C2), True)
    S2 = edge_phase(a0, a1, b0, b1, C2)
    (h,) = _k_node(S2, deg2d, xe2, pe2, node_W(L2, C2, None), False)

    onehot = (batch[:, None] == jnp.arange(NUM_GRAPHS)[None, :]).astype(jnp.float32)
    out, logits, topo_feat = _k_pool(h, onehot, params)
    return out, logits, topo_feat


# speedup vs baseline: 4.8565x; 1.2623x over previous
import functools

import jax
import jax.numpy as jnp
import numpy as np
from jax import lax
from jax.experimental import pallas as pl
from jax.experimental.pallas import tpu as pltpu
from jax.experimental.pallas import tpu_sc as plsc

N = 10000
E = 160000
H = 256
NUM_GRAPHS = 16
RADIUS = 4.0
SCALE_FACTORS = (1.0, 2.0, 4.0)

_SEG_NP = np.zeros((3, 256), np.float32)
_SEG_NP[0, :86] = 1.0
_SEG_NP[1, 86:171] = 1.0
_SEG_NP[2, 171:256] = 1.0

_NB = 1000
_EB = 2000

_NSUB = 16
_CH = 80
_EPT = E // _NSUB
_NCHUNK = _EPT // _CH
_CHP = 40
_EPT32 = E // 32
_NCHP = _EPT32 // _CHP
_ROWS_PT = 1000
_NDUMP = N // _ROWS_PT

_sc_mesh = lambda: plsc.VectorSubcoreMesh(core_axis_name="c", subcore_axis_name="s")


def _silu(x):
    return x * jax.nn.sigmoid(x)


def _ln(x, g, b):
    mu = jnp.mean(x, axis=-1, keepdims=True)
    var = jnp.mean((x - mu) ** 2, axis=-1, keepdims=True)
    return g * (x - mu) * jax.lax.rsqrt(var + 1e-5) + b


def _row(v):
    return v.reshape(1, -1)


def _sc_prologue(colp, rowp, pos128, zrows):
    @functools.partial(
        pl.kernel,
        mesh=_sc_mesh(),
        out_type=[
            jax.ShapeDtypeStruct((E, 16), jnp.float32),
            jax.ShapeDtypeStruct((2, N, 128), jnp.float32),
        ],
        scratch_types=[
            pltpu.VMEM((_CHP,), jnp.int32),
            pltpu.VMEM((_CHP,), jnp.int32),
            pltpu.VMEM((_CHP, 128), jnp.float32),
            pltpu.VMEM((_CHP, 128), jnp.float32),
            pltpu.VMEM((_CHP, 128), jnp.float32),
            pltpu.VMEM((_CHP, 16), jnp.float32),
            pltpu.VMEM_SHARED((N, 128), jnp.float32),
            pltpu.SemaphoreType.DMA,
            pltpu.SemaphoreType.DMA,
        ],
    )
    def k(col_h, row_h, pos_h, z_h, evec_o, deg_o,
          ci_v, ri_v, pr_v, pc_v, ones_v, vec_v, deg_sh, sem1, sem2):
        c = lax.axis_index("c")
        s = lax.axis_index("s")
        wid = c * _NSUB + s
        base = wid * _EPT32

        def fill(i, _):
            for j in range(8):
                ones_v[i, pl.ds(j * 16, 16)] = jnp.full((16,), 1.0, jnp.float32)
            return 0
        lax.fori_loop(0, _CHP, fill, 0)

        @pl.when(s < _NDUMP)
        def _():
            pltpu.sync_copy(z_h, deg_sh.at[pl.ds(s * _ROWS_PT, _ROWS_PT)])
        plsc.subcore_barrier()

        def chunk(kk, _):
            off = base + kk * _CHP
            pltpu.sync_copy(col_h.at[pl.ds(off, _CHP)], ci_v)
            pltpu.sync_copy(row_h.at[pl.ds(off, _CHP)], ri_v)
            cp1 = pltpu.async_copy(pos_h.at[ri_v], pr_v, sem1)
            cp2 = pltpu.async_copy(pos_h.at[ci_v], pc_v, sem2)
            cp1.wait()
            cp2.wait()

            def sub(i, _):
                vec_v[i, pl.ds(0, 16)] = (pr_v[i, pl.ds(0, 16)]
                                          - pc_v[i, pl.ds(0, 16)])
                return 0
            lax.fori_loop(0, _CHP, sub, 0)
            pltpu.sync_copy(vec_v, evec_o.at[pl.ds(off, _CHP)])
            pltpu.sync_copy(ones_v, deg_sh.at[ci_v], add=True)
            return 0
        lax.fori_loop(0, _NCHP, chunk, 0)

        plsc.subcore_barrier()

        @pl.when(s < _NDUMP)
        def _():
            pltpu.sync_copy(deg_sh.at[pl.ds(s * _ROWS_PT, _ROWS_PT)],
                            deg_o.at[c, pl.ds(s * _ROWS_PT, _ROWS_PT)])

    return k(colp, rowp, pos128, zrows)


def _sc_gather(col3d, row3d, ah0, ah1, bh0, bh1):
    @functools.partial(
        pl.kernel,
        mesh=_sc_mesh(),
        out_type=jax.ShapeDtypeStruct((2, E, 128), jnp.float32),
        scratch_types=[
            pltpu.VMEM((_NCHUNK, _CH), jnp.int32),
            pltpu.VMEM((_NCHUNK, _CH), jnp.int32),
            pltpu.VMEM((_CH, 128), jnp.float32),
            pltpu.VMEM((_CH, 128), jnp.float32),
            pltpu.VMEM((_CH, 128), jnp.float32),
            pltpu.VMEM((_CH, 128), jnp.float32),
            pltpu.SemaphoreType.DMA,
            pltpu.SemaphoreType.DMA,
            pltpu.SemaphoreType.DMA,
            pltpu.SemaphoreType.DMA,
        ],
    )
    def k(col_h, row_h, a0_h, a1_h, b0_h, b1_h, g_o,
          ci_v, ri_v, a_v0, b_v0, a_v1, b_v1, sa0, sb0, sa1, sb1):
        c = lax.axis_index("c")
        s = lax.axis_index("s")
        base = s * _EPT

        pltpu.sync_copy(col_h.at[s], ci_v)
        pltpu.sync_copy(row_h.at[s], ri_v)

        def fire(kk, a_v, b_v, sa, sb):
            @pl.when(c == 0)
            def _():
                pltpu.async_copy(a0_h.at[ci_v.at[kk]], a_v, sa)
                pltpu.async_copy(b0_h.at[ri_v.at[kk]], b_v, sb)

            @pl.when(c == 1)
            def _():
                pltpu.async_copy(a1_h.at[ci_v.at[kk]], a_v, sa)
                pltpu.async_copy(b1_h.at[ri_v.at[kk]], b_v, sb)

        def wait(a_v, b_v, sa, sb):
            pltpu.make_async_copy(a0_h.at[ci_v.at[0]], a_v, sa).wait()
            pltpu.make_async_copy(b0_h.at[ri_v.at[0]], b_v, sb).wait()

        def drain(kk, a_v, b_v, sa, sb):
            wait(a_v, b_v, sa, sb)

            def add(i, _):
                for j in range(8):
                    a_v[i, pl.ds(j * 16, 16)] = (a_v[i, pl.ds(j * 16, 16)]
                                                 + b_v[i, pl.ds(j * 16, 16)])
                return 0
            lax.fori_loop(0, _CH, add, 0)
            pltpu.sync_copy(a_v, g_o.at[c, pl.ds(base + kk * _CH, _CH)])

        fire(0, a_v0, b_v0, sa0, sb0)

        def pair(kk, _):
            fire(2 * kk + 1, a_v1, b_v1, sa1, sb1)
            drain(2 * kk, a_v0, b_v0, sa0, sb0)
            fire(2 * kk + 2, a_v0, b_v0, sa0, sb0)
            drain(2 * kk + 1, a_v1, b_v1, sa1, sb1)
            return 0
        lax.fori_loop(0, (_NCHUNK - 1) // 2, pair, 0)
        drain(_NCHUNK - 1, a_v0, b_v0, sa0, sb0)

    return k(col3d, row3d, ah0, ah1, bh0, bh1)


def _sc_scatter(colp, p0, p1, zrows):
    @functools.partial(
        pl.kernel,
        mesh=_sc_mesh(),
        out_type=jax.ShapeDtypeStruct((2, N, 128), jnp.float32),
        scratch_types=[
            pltpu.VMEM((_CH,), jnp.int32),
            pltpu.VMEM((_CH, 128), jnp.float32),
            pltpu.VMEM_SHARED((N, 128), jnp.float32),
            pltpu.SemaphoreType.DMA,
        ],
    )
    def k(col_h, p0_h, p1_h, z_h, s_o, ci_v, p_v, s_sh, sem1):
        c = lax.axis_index("c")
        s = lax.axis_index("s")
        base = s * _EPT

        @pl.when(s < _NDUMP)
        def _():
            pltpu.sync_copy(z_h, s_sh.at[pl.ds(s * _ROWS_PT, _ROWS_PT)])
        plsc.subcore_barrier()

        def chunk(kk, _):
            off = base + kk * _CH
            pltpu.sync_copy(col_h.at[pl.ds(off, _CH)], ci_v)

            @pl.when(c == 0)
            def _():
                pltpu.async_copy(p0_h.at[pl.ds(off, _CH)], p_v, sem1).wait()

            @pl.when(c == 1)
            def _():
                pltpu.async_copy(p1_h.at[pl.ds(off, _CH)], p_v, sem1).wait()

            pltpu.sync_copy(p_v, s_sh.at[ci_v], add=True)
            return 0
        lax.fori_loop(0, _NCHUNK, chunk, 0)

        plsc.subcore_barrier()

        @pl.when(s < _NDUMP)
        def _():
            pltpu.sync_copy(s_sh.at[pl.ds(s * _ROWS_PT, _ROWS_PT)],
                            s_o.at[c, pl.ds(s * _ROWS_PT, _ROWS_PT)])

    return k(colp, p0, p1, zrows)


def _pre_body(x_ref, pos_ref,
              win, bin_, gin, bln,
              p1a, p1ab, p1b, p1bb, p2a, p2ab, p2b, p2bb,
              wa1, wb1,
              xe1_o, a0_o, a1_o, b0_o, b1_o, pe2_o):
    x = x_ref[...]
    pos = pos_ref[...]
    h = _silu(_ln(jnp.dot(x, win[...], preferred_element_type=jnp.float32)
                  + bin_[...], gin[...], bln[...]))
    pe1 = jnp.dot(_silu(jnp.dot(pos, p1a[...], preferred_element_type=jnp.float32)
                        + p1ab[...]), p1b[...],
                  preferred_element_type=jnp.float32) + p1bb[...]
    pe2 = jnp.dot(_silu(jnp.dot(pos, p2a[...], preferred_element_type=jnp.float32)
                        + p2ab[...]), p2b[...],
                  preferred_element_type=jnp.float32) + p2bb[...]
    xe1 = h + pe1
    xe1_o[...] = xe1
    pe2_o[...] = pe2
    a = jnp.dot(xe1, wa1[...], preferred_element_type=jnp.float32)
    b = jnp.dot(xe1, wb1[...], preferred_element_type=jnp.float32)
    a0_o[...] = a[:, :128]
    a1_o[...] = a[:, 128:]
    b0_o[...] = b[:, :128]
    b1_o[...] = b[:, 128:]


def _k_pre(x, pos8, W):
    grid = (N // _NB,)
    bspec_n = pl.BlockSpec((_NB, 256), lambda i: (i, 0))
    bspec_h = pl.BlockSpec((_NB, 128), lambda i: (i, 0))
    bspec_p = pl.BlockSpec((_NB, 8), lambda i: (i, 0))
    cw = lambda shape: pl.BlockSpec(shape, lambda i: tuple(0 for _ in shape))
    out_shapes = ([jax.ShapeDtypeStruct((N, 256), jnp.float32)]
                  + [jax.ShapeDtypeStruct((N, 128), jnp.float32)] * 4
                  + [jax.ShapeDtypeStruct((N, 256), jnp.float32)])
    return pl.pallas_call(
        _pre_body,
        grid=grid,
        in_specs=[bspec_n, bspec_p,
                  cw((256, 256)), cw((1, 256)), cw((1, 256)), cw((1, 256)),
                  cw((8, 128)), cw((1, 128)), cw((128, 256)), cw((1, 256)),
                  cw((8, 128)), cw((1, 128)), cw((128, 256)), cw((1, 256)),
                  cw((256, 256)), cw((256, 256))],
        out_specs=[bspec_n, bspec_h, bspec_h, bspec_h, bspec_h, bspec_n],
        out_shape=out_shapes,
    )(x, pos8, *W)


def _ea_body(ev_ref, ea_o):
    ev = ev_ref[...][:, :3]
    ed = jnp.sqrt(jnp.sum(ev * ev, axis=-1, keepdims=True))
    dirn = ev / (ed + 1e-8)
    dn = ed / (RADIUS + 1e-8)
    z = jnp.zeros_like(ev_ref[...][:, :4])
    ea_o[...] = jnp.concatenate([dirn, dn, z], axis=-1)


def _k_ea(evec):
    grid = (E // _EB,)
    bspec16 = pl.BlockSpec((_EB, 16), lambda i: (i, 0))
    bspec8 = pl.BlockSpec((_EB, 8), lambda i: (i, 0))
    return pl.pallas_call(
        _ea_body,
        grid=grid,
        in_specs=[bspec16],
        out_specs=bspec8,
        out_shape=jax.ShapeDtypeStruct((E, 8), jnp.float32),
    )(evec)


def _edge_body(g0, g1, ea, wc, c0, seg, gg, bb, p0_o, p1_o):
    G = jnp.concatenate([g0[0], g1[0]], axis=-1)
    G = G + jnp.dot(ea[...], wc[...], preferred_element_type=jnp.float32) + c0[...]
    segm = seg[...]
    dims = jnp.sum(segm, axis=1, keepdims=True).T
    mu = jnp.dot(G, segm.T, preferred_element_type=jnp.float32) / dims
    muf = jnp.dot(mu, segm, preferred_element_type=jnp.float32)
    cen = G - muf
    var = jnp.dot(cen * cen, segm.T, preferred_element_type=jnp.float32) / dims
    denom = jax.lax.rsqrt(jnp.dot(var, segm, preferred_element_type=jnp.float32) + 1e-5)
    P = _silu(gg[...] * cen * denom + bb[...])
    p0_o[...] = P[:, :128]
    p1_o[...] = P[:, 128:]


def _k_edge(G, ea8, wc, c0, seg, gcat, bcat):
    grid = (E // _EB,)
    h0 = pl.BlockSpec((1, _EB, 128), lambda i: (0, i, 0))
    h1 = pl.BlockSpec((1, _EB, 128), lambda i: (1, i, 0))
    be = pl.BlockSpec((_EB, 8), lambda i: (i, 0))
    bh = pl.BlockSpec((_EB, 128), lambda i: (i, 0))
    cw = lambda shape: pl.BlockSpec(shape, lambda i: tuple(0 for _ in shape))
    return pl.pallas_call(
        _edge_body,
        grid=grid,
        in_specs=[h0, h1, be,
                  cw((8, 256)), cw((1, 256)), cw((3, 256)), cw((1, 256)), cw((1, 256))],
        out_specs=[bh, bh],
        out_shape=[jax.ShapeDtypeStruct((E, 128), jnp.float32)] * 2,
    )(G, G, ea8, wc, c0, seg, gcat, bcat)


def _node_body(has_next, s0, s1, deg_ref, xe_ref, pen_ref,
               w2bd, b2, seg, ws1, bs1, ws2, bs2,
               wu1x, wu1w, bu1, gu, bu, wu2, bu2, gn, bn,
               wan, wbn, *outs):
    S = jnp.concatenate([s0[0], s1[0]], axis=-1)
    xe = xe_ref[...]
    agg = (jnp.dot(S, w2bd[...], preferred_element_type=jnp.float32)
           + deg_ref[...] * b2[...])
    t = _silu(jnp.dot(agg, ws1[...], preferred_element_type=jnp.float32) + bs1[...])
    logit = jnp.dot(t, ws2[...], preferred_element_type=jnp.float32) + bs2[...]
    aw = jax.nn.softmax(logit, axis=-1)
    weighted = agg * jnp.dot(aw, seg[...], preferred_element_type=jnp.float32)
    u = (jnp.dot(xe, wu1x[...], preferred_element_type=jnp.float32)
         + jnp.dot(weighted, wu1w[...], preferred_element_type=jnp.float32)
         + bu1[...])
    u = _silu(_ln(u, gu[...], bu[...]))
    u = jnp.dot(u, wu2[...], preferred_element_type=jnp.float32) + bu2[...]
    h = _ln(u + xe, gn[...], bn[...])
    if has_next:
        xe2_o, a0_o, a1_o, b0_o, b1_o = outs
        xe2 = h + pen_ref[...]
        xe2_o[...] = xe2
        a = jnp.dot(xe2, wan[...], preferred_element_type=jnp.float32)
        b = jnp.dot(xe2, wbn[...], preferred_element_type=jnp.float32)
        a0_o[...] = a[:, :128]
        a1_o[...] = a[:, 128:]
        b0_o[...] = b[:, :128]
        b1_o[...] = b[:, 128:]
    else:
        outs[0][...] = h


def _k_node(S2h, deg2d, xe, pe_next, W, has_next):
    grid = (N // _NB,)
    bspec_n = pl.BlockSpec((_NB, 256), lambda i: (i, 0))
    bspec_h = pl.BlockSpec((_NB, 128), lambda i: (i, 0))
    bspec_d = pl.BlockSpec((_NB, 1), lambda i: (i, 0))
    s0 = pl.BlockSpec((1, _NB, 128), lambda i: (0, i, 0))
    s1 = pl.BlockSpec((1, _NB, 128), lambda i: (1, i, 0))
    cw = lambda shape: pl.BlockSpec(shape, lambda i: tuple(0 for _ in shape))
    w_specs = [cw((256, 256)), cw((1, 256)), cw((3, 256)),
               cw((256, 64)), cw((1, 64)), cw((64, 3)), cw((1, 3)),
               cw((256, 512)), cw((256, 512)), cw((1, 512)), cw((1, 512)), cw((1, 512)),
               cw((512, 256)), cw((1, 256)), cw((1, 256)), cw((1, 256)),
               cw((256, 256)), cw((256, 256))]
    if has_next:
        out_specs = [bspec_n, bspec_h, bspec_h, bspec_h, bspec_h]
        out_shape = ([jax.ShapeDtypeStruct((N, 256), jnp.float32)]
                     + [jax.ShapeDtypeStruct((N, 128), jnp.float32)] * 4)
    else:
        out_specs = [bspec_n]
        out_shape = [jax.ShapeDtypeStruct((N, 256), jnp.float32)]
    return pl.pallas_call(
        functools.partial(_node_body, has_next),
        grid=grid,
        in_specs=[s0, s1, bspec_d, bspec_n, bspec_n] + w_specs,
        out_specs=out_specs,
        out_shape=out_shape,
    )(S2h, S2h, deg2d, xe, pe_next, *W)


def _pool_body(h_ref, oh_ref,
               wa1, ba1, wa2, ba2,
               wpf, bpf, gpf, bbpf,
               wc1, bc1, wc2, bc2,
               wz1, bz1, wz2, bz2,
               wg1, bg1, wg2, bg2,
               ws1_, bs1_, ws2_, bs2_,
               wo1, bo1, go1, bo1n, wo2, bo2, go2, bo2n, wo3, bo3,
               wt1, bt1, wt2, bt2,
               out_o, log_o, topo_o,
               msum, mxp, attn, cnt, mse):
    i = pl.program_id(0)

    @pl.when(i == 0)
    def _():
        msum[...] = jnp.zeros_like(msum)
        mxp[...] = jnp.full_like(mxp, -3e38)
        attn[...] = jnp.zeros_like(attn)
        cnt[...] = jnp.zeros_like(cnt)
        mse[...] = jnp.concatenate(
            [jnp.full((1, 1), -3e38, jnp.float32),
             jnp.zeros((1, 1), jnp.float32)], axis=-1)

    h = h_ref[...]
    oh = oh_ref[...]
    ohT_dot = lambda rhs: jax.lax.dot_general(
        oh, rhs, (((0,), (0,)), ((), ())), preferred_element_type=jnp.float32)

    s = (jnp.dot(_silu(jnp.dot(h, wa1[...], preferred_element_type=jnp.float32)
                       + ba1[...]), wa2[...],
                 preferred_element_type=jnp.float32) + ba2[...])
    m_old = mse[0, 0]
    se_old = mse[0, 1]
    m_new = jnp.maximum(m_old, jnp.max(s))
    scale = jnp.exp(m_old - m_new)
    e = jnp.exp(s - m_new)
    se_new = se_old * scale + jnp.sum(e)
    mse[...] = jnp.concatenate([jnp.full((1, 1), m_new, jnp.float32),
                                jnp.full((1, 1), se_new, jnp.float32)], axis=-1)
    attn[...] = attn[...] * scale + ohT_dot(h * e)
    msum[...] = msum[...] + ohT_dot(h)
    cnt[...] = cnt[...] + ohT_dot(jnp.ones_like(s))
    rows = [jnp.max(jnp.where(oh[:, g:g + 1] > 0.0, h, -3e38), axis=0,
                    keepdims=True) for g in range(NUM_GRAPHS)]
    mxp[...] = jnp.maximum(mxp[...], jnp.concatenate(rows, axis=0))

    mean = msum[...] / jnp.maximum(cnt[...], 1.0)
    att_pool = attn[...] / mse[0, 1]
    combined = jnp.concatenate([mean, mxp[...], att_pool], axis=-1)
    pooled = _silu(_ln(jnp.dot(combined, wpf[...],
                               preferred_element_type=jnp.float32) + bpf[...],
                       gpf[...], bbpf[...]))

    def head(w1, b1, w2, b2):
        t = _silu(jnp.dot(pooled, w1[...], preferred_element_type=jnp.float32)
                  + b1[...])
        return jnp.dot(t, w2[...], preferred_element_type=jnp.float32) + b2[...]

    chern = jnp.tanh(head(wc1, bc1, wc2, bc2))
    z2 = jax.nn.sigmoid(head(wz1, bz1, wz2, bz2))
    gap = jax.nn.softplus(head(wg1, bg1, wg2, bg2))
    sym = head(ws1_, bs1_, ws2_, bs2_)
    topo = jnp.concatenate([chern, z2, gap, sym], axis=-1)
    f = jnp.concatenate([pooled, topo], axis=-1)
    f = _silu(_ln(jnp.dot(f, wo1[...], preferred_element_type=jnp.float32)
                  + bo1[...], go1[...], bo1n[...]))
    f = _silu(_ln(jnp.dot(f, wo2[...], preferred_element_type=jnp.float32)
                  + bo2[...], go2[...], bo2n[...]))
    out = jnp.dot(f, wo3[...], preferred_element_type=jnp.float32) + bo3[...]
    t2 = _silu(jnp.dot(out, wt1[...], preferred_element_type=jnp.float32)
               + bt1[...])
    logits = jnp.dot(t2, wt2[...], preferred_element_type=jnp.float32) + bt2[...]
    out_o[...] = out
    log_o[...] = logits
    topo_o[...] = topo


def _k_pool(h, onehot, params):
    ap = params['att_pool']
    pf = params['pool_fusion']
    t = params['topo']
    op = params['output_proj']
    th = params['topo_head']
    W = [
        ap['l1']['w'], _row(ap['l1']['b']), ap['l2']['w'], _row(ap['l2']['b']),
        pf['lin']['w'], _row(pf['lin']['b']), _row(pf['ln']['g']), _row(pf['ln']['b']),
        t['chern']['l1']['w'], _row(t['chern']['l1']['b']),
        t['chern']['l2']['w'], _row(t['chern']['l2']['b']),
        t['z2']['l1']['w'], _row(t['z2']['l1']['b']),
        t['z2']['l2']['w'], _row(t['z2']['l2']['b']),
        t['gap']['l1']['w'], _row(t['gap']['l1']['b']),
        t['gap']['l2']['w'], _row(t['gap']['l2']['b']),
        t['sym']['l1']['w'], _row(t['sym']['l1']['b']),
        t['sym']['l2']['w'], _row(t['sym']['l2']['b']),
        op['l1']['w'], _row(op['l1']['b']), _row(op['ln1']['g']), _row(op['ln1']['b']),
        op['l2']['w'], _row(op['l2']['b']), _row(op['ln2']['g']), _row(op['ln2']['b']),
        op['l3']['w'], _row(op['l3']['b']),
        th['l1']['w'], _row(th['l1']['b']), th['l2']['w'], _row(th['l2']['b']),
    ]
    grid = (N // _NB,)
    bspec_n = pl.BlockSpec((_NB, 256), lambda i: (i, 0))
    bspec_o = pl.BlockSpec((_NB, 16), lambda i: (i, 0))
    cw = lambda a: pl.BlockSpec(a.shape, lambda i: tuple(0 for _ in a.shape))
    return pl.pallas_call(
        _pool_body,
        grid=grid,
        in_specs=[bspec_n, bspec_o] + [cw(w) for w in W],
        out_specs=[pl.BlockSpec((16, 128), lambda i: (0, 0)),
                   pl.BlockSpec((16, 3), lambda i: (0, 0)),
                   pl.BlockSpec((16, 14), lambda i: (0, 0))],
        out_shape=[jax.ShapeDtypeStruct((16, 128), jnp.float32),
                   jax.ShapeDtypeStruct((16, 3), jnp.float32),
                   jax.ShapeDtypeStruct((16, 14), jnp.float32)],
        scratch_shapes=[pltpu.VMEM((16, 256), jnp.float32),
                        pltpu.VMEM((16, 256), jnp.float32),
                        pltpu.VMEM((16, 256), jnp.float32),
                        pltpu.VMEM((16, 1), jnp.float32),
                        pltpu.VMEM((1, 2), jnp.float32)],
    )(h, onehot, *W)


def _layer_consts(lp):
    WAs, WBs, Wcs, c0s, gc, bc, b2c = [], [], [], [], [], [], []
    blocks = []
    for i, s in enumerate(SCALE_FACTORS):
        mp = lp['msg'][i]
        W1 = mp['l1']['w']
        WAs.append(W1[:256])
        WBs.append(W1[256:512])
        Wcs.append(W1[512:516])
        c0s.append(s * W1[516] + mp['l1']['b'])
        gc.append(mp['ln']['g'])
        bc.append(mp['ln']['b'])
        blocks.append(mp['l2']['w'])
        b2c.append(mp['l2']['b'])
    W2bd = jax.scipy.linalg.block_diag(*blocks)
    Wc = jnp.concatenate(Wcs, 1)
    Wc8 = jnp.pad(Wc, ((0, 4), (0, 0)))
    return dict(
        WA=jnp.concatenate(WAs, 1), WB=jnp.concatenate(WBs, 1),
        Wc8=Wc8, c0=jnp.concatenate(c0s),
        g=jnp.concatenate(gc), b=jnp.concatenate(bc),
        W2bd=W2bd, b2=jnp.concatenate(b2c))


def kernel(x, pos, edge_index, batch, params):
    seg = jnp.asarray(_SEG_NP)
    row = edge_index[0].astype(jnp.int32)
    col = edge_index[1].astype(jnp.int32)

    L1, L2 = params['layers'][0], params['layers'][1]
    C1, C2 = _layer_consts(L1), _layer_consts(L2)

    pos128 = jnp.pad(pos, ((0, 0), (0, 125)))
    zrows = jnp.zeros((_ROWS_PT, 128), jnp.float32)

    evec, degp = _sc_prologue(col, row, pos128, zrows)
    deg2d = (degp[0, :, 0] + degp[1, :, 0])[:, None]
    ea8 = _k_ea(evec)

    pos8 = jnp.pad(pos, ((0, 0), (0, 5)))
    ip = params['input_proj']
    pe_w = lambda lp, k: lp['pos_enc'][k]
    pre_W = [
        ip['lin']['w'], _row(ip['lin']['b']), _row(ip['ln']['g']), _row(ip['ln']['b']),
        jnp.pad(pe_w(L1, 'l1')['w'], ((0, 5), (0, 0))), _row(pe_w(L1, 'l1')['b']),
        pe_w(L1, 'l2')['w'], _row(pe_w(L1, 'l2')['b']),
        jnp.pad(pe_w(L2, 'l1')['w'], ((0, 5), (0, 0))), _row(pe_w(L2, 'l1')['b']),
        pe_w(L2, 'l2')['w'], _row(pe_w(L2, 'l2')['b']),
        C1['WA'], C1['WB'],
    ]
    xe1, a0, a1, b0, b1, pe2 = _k_pre(x, pos8, pre_W)

    col3d = col.reshape(_NSUB, _NCHUNK, _CH)
    row3d = row.reshape(_NSUB, _NCHUNK, _CH)

    def edge_phase(a0, a1, b0, b1, C):
        G = _sc_gather(col3d, row3d, a0, a1, b0, b1)
        P0, P1 = _k_edge(G, ea8, C['Wc8'], _row(C['c0']), seg,
                         _row(C['g']), _row(C['b']))
        return _sc_scatter(col, P0, P1, zrows)

    def node_W(lp, C, Cn):
        up = lp['update']
        sa = lp['scale_att']
        Wu1 = up['l1']['w']
        return [
            C['W2bd'], _row(C['b2']), seg,
            sa['l1']['w'], _row(sa['l1']['b']), sa['l2']['w'], _row(sa['l2']['b']),
            Wu1[:256], Wu1[256:], _row(up['l1']['b']),
            _row(up['ln']['g']), _row(up['ln']['b']),
            up['l2']['w'], _row(up['l2']['b']),
            _row(lp['norm']['g']), _row(lp['norm']['b']),
            (Cn['WA'] if Cn is not None else C['WA']),
            (Cn['WB'] if Cn is not None else C['WB']),
        ]

    S1 = edge_phase(a0, a1, b0, b1, C1)
    xe2, a0, a1, b0, b1 = _k_node(S1, deg2d, xe1, pe2, node_W(L1, C1, C2), True)
    S2 = edge_phase(a0, a1, b0, b1, C2)
    (h,) = _k_node(S2, deg2d, xe2, pe2, node_W(L2, C2, None), False)

    onehot = (batch[:, None] == jnp.arange(NUM_GRAPHS)[None, :]).astype(jnp.float32)
    out, logits, topo_feat = _k_pool(h, onehot, params)
    return out, logits, topo_feat


# speedup vs baseline: 6.5866x; 1.3562x over previous
import functools

import jax
import jax.numpy as jnp
import numpy as np
from jax import lax
from jax.experimental import pallas as pl
from jax.experimental.pallas import tpu as pltpu
from jax.experimental.pallas import tpu_sc as plsc

N = 10000
E = 160000
H = 256
NUM_GRAPHS = 16
RADIUS = 4.0
SCALE_FACTORS = (1.0, 2.0, 4.0)

_SEG_NP = np.zeros((3, 256), np.float32)
_SEG_NP[0, :86] = 1.0
_SEG_NP[1, 86:171] = 1.0
_SEG_NP[2, 171:256] = 1.0

_NB = 1000
_EB = 2000

_NSUB = 16
_CH = 80
_EPT = E // _NSUB
_NCHUNK = _EPT // _CH
_CHP = 40
_EPT32 = E // 32
_NCHP = _EPT32 // _CHP
_ROWS_PT = 1000
_NDUMP = N // _ROWS_PT

_sc_mesh = lambda: plsc.VectorSubcoreMesh(core_axis_name="c", subcore_axis_name="s")


def _silu(x):
    return x * jax.nn.sigmoid(x)


def _ln(x, g, b):
    mu = jnp.mean(x, axis=-1, keepdims=True)
    var = jnp.mean((x - mu) ** 2, axis=-1, keepdims=True)
    return g * (x - mu) * jax.lax.rsqrt(var + 1e-5) + b


def _row(v):
    return v.reshape(1, -1)


def _sc_prologue(colp, rowp, pos128):
    @functools.partial(
        pl.kernel,
        mesh=_sc_mesh(),
        out_type=jax.ShapeDtypeStruct((E, 16), jnp.float32),
        scratch_types=[
            pltpu.VMEM((_NCHP, _CHP), jnp.int32),
            pltpu.VMEM((_NCHP, _CHP), jnp.int32),
            pltpu.VMEM((_CHP, 128), jnp.float32),
            pltpu.VMEM((_CHP, 128), jnp.float32),
            pltpu.VMEM((_CHP, 128), jnp.float32),
            pltpu.VMEM((_CHP, 128), jnp.float32),
            pltpu.VMEM((_CHP, 16), jnp.float32),
            pltpu.SemaphoreType.DMA,
            pltpu.SemaphoreType.DMA,
            pltpu.SemaphoreType.DMA,
            pltpu.SemaphoreType.DMA,
        ],
    )
    def k(col_h, row_h, pos_h, evec_o,
          ci_v, ri_v, pr_v0, pc_v0, pr_v1, pc_v1, vec_v,
          sr0, sc0, sr1, sc1):
        c = lax.axis_index("c")
        s = lax.axis_index("s")
        wid = c * _NSUB + s
        base = wid * _EPT32

        pltpu.sync_copy(col_h.at[wid], ci_v)
        pltpu.sync_copy(row_h.at[wid], ri_v)

        def fire(kk, pr_v, pc_v, sr, sc):
            pltpu.async_copy(pos_h.at[ri_v.at[kk]], pr_v, sr)
            pltpu.async_copy(pos_h.at[ci_v.at[kk]], pc_v, sc)

        def drain(kk, pr_v, pc_v, sr, sc):
            pltpu.make_async_copy(pos_h.at[ri_v.at[0]], pr_v, sr).wait()
            pltpu.make_async_copy(pos_h.at[ci_v.at[0]], pc_v, sc).wait()

            def sub(i, _):
                vec_v[i, pl.ds(0, 16)] = (pr_v[i, pl.ds(0, 16)]
                                          - pc_v[i, pl.ds(0, 16)])
                return 0
            lax.fori_loop(0, _CHP, sub, 0)
            pltpu.sync_copy(vec_v, evec_o.at[pl.ds(base + kk * _CHP, _CHP)])

        fire(0, pr_v0, pc_v0, sr0, sc0)

        def pair(kk, _):
            fire(2 * kk + 1, pr_v1, pc_v1, sr1, sc1)
            drain(2 * kk, pr_v0, pc_v0, sr0, sc0)
            fire(2 * kk + 2, pr_v0, pc_v0, sr0, sc0)
            drain(2 * kk + 1, pr_v1, pc_v1, sr1, sc1)
            return 0
        lax.fori_loop(0, (_NCHP - 1) // 2, pair, 0)
        drain(_NCHP - 1, pr_v0, pc_v0, sr0, sc0)

    return k(colp, rowp, pos128)


def _sc_deg(col3d, zrows):
    @functools.partial(
        pl.kernel,
        mesh=_sc_mesh(),
        out_type=jax.ShapeDtypeStruct((2, N, 128), jnp.float32),
        scratch_types=[
            pltpu.VMEM((_NCHP, _CHP), jnp.int32),
            pltpu.VMEM((_CHP, 128), jnp.float32),
            pltpu.VMEM_SHARED((N, 128), jnp.float32),
        ],
    )
    def k(col_h, z_h, deg_o, ci_v, ones_v, deg_sh):
        c = lax.axis_index("c")
        s = lax.axis_index("s")
        wid = c * _NSUB + s

        pltpu.sync_copy(col_h.at[wid], ci_v)

        def fill(i, _):
            for j in range(8):
                ones_v[i, pl.ds(j * 16, 16)] = jnp.full((16,), 1.0, jnp.float32)
            return 0
        lax.fori_loop(0, _CHP, fill, 0)

        @pl.when(s < _NDUMP)
        def _():
            pltpu.sync_copy(z_h, deg_sh.at[pl.ds(s * _ROWS_PT, _ROWS_PT)])
        plsc.subcore_barrier()

        def chunk(kk, _):
            pltpu.sync_copy(ones_v, deg_sh.at[ci_v.at[kk]], add=True)
            return 0
        lax.fori_loop(0, _NCHP, chunk, 0)

        plsc.subcore_barrier()

        @pl.when(s < _NDUMP)
        def _():
            pltpu.sync_copy(deg_sh.at[pl.ds(s * _ROWS_PT, _ROWS_PT)],
                            deg_o.at[c, pl.ds(s * _ROWS_PT, _ROWS_PT)])

    return k(col3d, zrows)


def _sc_gather(col3d, row3d, ah0, ah1, bh0, bh1):
    @functools.partial(
        pl.kernel,
        mesh=_sc_mesh(),
        out_type=jax.ShapeDtypeStruct((2, E, 128), jnp.float32),
        scratch_types=[
            pltpu.VMEM((_NCHUNK, _CH), jnp.int32),
            pltpu.VMEM((_NCHUNK, _CH), jnp.int32),
            pltpu.VMEM((_CH, 128), jnp.float32),
            pltpu.VMEM((_CH, 128), jnp.float32),
            pltpu.VMEM((_CH, 128), jnp.float32),
            pltpu.VMEM((_CH, 128), jnp.float32),
            pltpu.SemaphoreType.DMA,
            pltpu.SemaphoreType.DMA,
            pltpu.SemaphoreType.DMA,
            pltpu.SemaphoreType.DMA,
        ],
    )
    def k(col_h, row_h, a0_h, a1_h, b0_h, b1_h, g_o,
          ci_v, ri_v, a_v0, b_v0, a_v1, b_v1, sa0, sb0, sa1, sb1):
        c = lax.axis_index("c")
        s = lax.axis_index("s")
        base = s * _EPT

        pltpu.sync_copy(col_h.at[s], ci_v)
        pltpu.sync_copy(row_h.at[s], ri_v)

        def fire(kk, a_v, b_v, sa, sb):
            @pl.when(c == 0)
            def _():
                pltpu.async_copy(a0_h.at[ci_v.at[kk]], a_v, sa)
                pltpu.async_copy(b0_h.at[ri_v.at[kk]], b_v, sb)

            @pl.when(c == 1)
            def _():
                pltpu.async_copy(a1_h.at[ci_v.at[kk]], a_v, sa)
                pltpu.async_copy(b1_h.at[ri_v.at[kk]], b_v, sb)

        def wait(a_v, b_v, sa, sb):
            pltpu.make_async_copy(a0_h.at[ci_v.at[0]], a_v, sa).wait()
            pltpu.make_async_copy(b0_h.at[ri_v.at[0]], b_v, sb).wait()

        def drain(kk, a_v, b_v, sa, sb):
            wait(a_v, b_v, sa, sb)

            def add(i, _):
                for j in range(8):
                    a_v[i, pl.ds(j * 16, 16)] = (a_v[i, pl.ds(j * 16, 16)]
                                                 + b_v[i, pl.ds(j * 16, 16)])
                return 0
            lax.fori_loop(0, _CH, add, 0)
            pltpu.sync_copy(a_v, g_o.at[c, pl.ds(base + kk * _CH, _CH)])

        fire(0, a_v0, b_v0, sa0, sb0)

        def pair(kk, _):
            fire(2 * kk + 1, a_v1, b_v1, sa1, sb1)
            drain(2 * kk, a_v0, b_v0, sa0, sb0)
            fire(2 * kk + 2, a_v0, b_v0, sa0, sb0)
            drain(2 * kk + 1, a_v1, b_v1, sa1, sb1)
            return 0
        lax.fori_loop(0, (_NCHUNK - 1) // 2, pair, 0)
        drain(_NCHUNK - 1, a_v0, b_v0, sa0, sb0)

    return k(col3d, row3d, ah0, ah1, bh0, bh1)


def _sc_scatter(col3d, p0, p1, zrows):
    @functools.partial(
        pl.kernel,
        mesh=_sc_mesh(),
        out_type=jax.ShapeDtypeStruct((2, N, 128), jnp.float32),
        scratch_types=[
            pltpu.VMEM((_NCHUNK, _CH), jnp.int32),
            pltpu.VMEM((_CH, 128), jnp.float32),
            pltpu.VMEM((_CH, 128), jnp.float32),
            pltpu.VMEM_SHARED((N, 128), jnp.float32),
            pltpu.SemaphoreType.DMA,
            pltpu.SemaphoreType.DMA,
        ],
    )
    def k(col_h, p0_h, p1_h, z_h, s_o, ci_v, p_v0, p_v1, s_sh, s0, s1):
        c = lax.axis_index("c")
        s = lax.axis_index("s")
        base = s * _EPT

        pltpu.sync_copy(col_h.at[s], ci_v)

        @pl.when(s < _NDUMP)
        def _():
            pltpu.sync_copy(z_h, s_sh.at[pl.ds(s * _ROWS_PT, _ROWS_PT)])
        plsc.subcore_barrier()

        def fire(kk, p_v, sem):
            off = base + kk * _CH

            @pl.when(c == 0)
            def _():
                pltpu.async_copy(p0_h.at[pl.ds(off, _CH)], p_v, sem)

            @pl.when(c == 1)
            def _():
                pltpu.async_copy(p1_h.at[pl.ds(off, _CH)], p_v, sem)

        def drain(kk, p_v, sem):
            pltpu.make_async_copy(p0_h.at[pl.ds(0, _CH)], p_v, sem).wait()
            pltpu.sync_copy(p_v, s_sh.at[ci_v.at[kk]], add=True)

        fire(0, p_v0, s0)

        def pair(kk, _):
            fire(2 * kk + 1, p_v1, s1)
            drain(2 * kk, p_v0, s0)
            fire(2 * kk + 2, p_v0, s0)
            drain(2 * kk + 1, p_v1, s1)
            return 0
        lax.fori_loop(0, (_NCHUNK - 1) // 2, pair, 0)
        drain(_NCHUNK - 1, p_v0, s0)

        plsc.subcore_barrier()

        @pl.when(s < _NDUMP)
        def _():
            pltpu.sync_copy(s_sh.at[pl.ds(s * _ROWS_PT, _ROWS_PT)],
                            s_o.at[c, pl.ds(s * _ROWS_PT, _ROWS_PT)])

    return k(col3d, p0, p1, zrows)


def _pre_body(x_ref, pos_ref,
              win, bin_, gin, bln,
              p1a, p1ab, p1b, p1bb, p2a, p2ab, p2b, p2bb,
              wa1, wb1,
              xe1_o, a0_o, a1_o, b0_o, b1_o, pe2_o):
    x = x_ref[...]
    pos = pos_ref[...]
    h = _silu(_ln(jnp.dot(x, win[...], preferred_element_type=jnp.float32)
                  + bin_[...], gin[...], bln[...]))
    pe1 = jnp.dot(_silu(jnp.dot(pos, p1a[...], preferred_element_type=jnp.float32)
                        + p1ab[...]), p1b[...],
                  preferred_element_type=jnp.float32) + p1bb[...]
    pe2 = jnp.dot(_silu(jnp.dot(pos, p2a[...], preferred_element_type=jnp.float32)
                        + p2ab[...]), p2b[...],
                  preferred_element_type=jnp.float32) + p2bb[...]
    xe1 = h + pe1
    xe1_o[...] = xe1
    pe2_o[...] = pe2
    a = jnp.dot(xe1, wa1[...], preferred_element_type=jnp.float32)
    b = jnp.dot(xe1, wb1[...], preferred_element_type=jnp.float32)
    a0_o[...] = a[:, :128]
    a1_o[...] = a[:, 128:]
    b0_o[...] = b[:, :128]
    b1_o[...] = b[:, 128:]


def _k_pre(x, pos8, W):
    grid = (N // _NB,)
    bspec_n = pl.BlockSpec((_NB, 256), lambda i: (i, 0))
    bspec_h = pl.BlockSpec((_NB, 128), lambda i: (i, 0))
    bspec_p = pl.BlockSpec((_NB, 8), lambda i: (i, 0))
    cw = lambda shape: pl.BlockSpec(shape, lambda i: tuple(0 for _ in shape))
    out_shapes = ([jax.ShapeDtypeStruct((N, 256), jnp.float32)]
                  + [jax.ShapeDtypeStruct((N, 128), jnp.float32)] * 4
                  + [jax.ShapeDtypeStruct((N, 256), jnp.float32)])
    return pl.pallas_call(
        _pre_body,
        grid=grid,
        in_specs=[bspec_n, bspec_p,
                  cw((256, 256)), cw((1, 256)), cw((1, 256)), cw((1, 256)),
                  cw((8, 128)), cw((1, 128)), cw((128, 256)), cw((1, 256)),
                  cw((8, 128)), cw((1, 128)), cw((128, 256)), cw((1, 256)),
                  cw((256, 256)), cw((256, 256))],
        out_specs=[bspec_n, bspec_h, bspec_h, bspec_h, bspec_h, bspec_n],
        out_shape=out_shapes,
    )(x, pos8, *W)


def _ea_body(ev_ref, ea_o):
    ev = ev_ref[...][:, :3]
    ed = jnp.sqrt(jnp.sum(ev * ev, axis=-1, keepdims=True))
    dirn = ev / (ed + 1e-8)
    dn = ed / (RADIUS + 1e-8)
    z = jnp.zeros_like(ev_ref[...][:, :4])
    ea_o[...] = jnp.concatenate([dirn, dn, z], axis=-1)


def _k_ea(evec):
    grid = (E // _EB,)
    bspec16 = pl.BlockSpec((_EB, 16), lambda i: (i, 0))
    bspec8 = pl.BlockSpec((_EB, 8), lambda i: (i, 0))
    return pl.pallas_call(
        _ea_body,
        grid=grid,
        in_specs=[bspec16],
        out_specs=bspec8,
        out_shape=jax.ShapeDtypeStruct((E, 8), jnp.float32),
    )(evec)


def _edge_body(g0, g1, ea, wc, c0, seg, gg, bb, p0_o, p1_o):
    G = jnp.concatenate([g0[0], g1[0]], axis=-1)
    G = G + jnp.dot(ea[...], wc[...], preferred_element_type=jnp.float32) + c0[...]
    segm = seg[...]
    dims = jnp.sum(segm, axis=1, keepdims=True).T
    mu = jnp.dot(G, segm.T, preferred_element_type=jnp.float32) / dims
    muf = jnp.dot(mu, segm, preferred_element_type=jnp.float32)
    cen = G - muf
    var = jnp.dot(cen * cen, segm.T, preferred_element_type=jnp.float32) / dims
    denom = jax.lax.rsqrt(jnp.dot(var, segm, preferred_element_type=jnp.float32) + 1e-5)
    P = _silu(gg[...] * cen * denom + bb[...])
    p0_o[...] = P[:, :128]
    p1_o[...] = P[:, 128:]


def _k_edge(G, ea8, wc, c0, seg, gcat, bcat):
    grid = (E // _EB,)
    h0 = pl.BlockSpec((1, _EB, 128), lambda i: (0, i, 0))
    h1 = pl.BlockSpec((1, _EB, 128), lambda i: (1, i, 0))
    be = pl.BlockSpec((_EB, 8), lambda i: (i, 0))
    bh = pl.BlockSpec((_EB, 128), lambda i: (i, 0))
    cw = lambda shape: pl.BlockSpec(shape, lambda i: tuple(0 for _ in shape))
    return pl.pallas_call(
        _edge_body,
        grid=grid,
        in_specs=[h0, h1, be,
                  cw((8, 256)), cw((1, 256)), cw((3, 256)), cw((1, 256)), cw((1, 256))],
        out_specs=[bh, bh],
        out_shape=[jax.ShapeDtypeStruct((E, 128), jnp.float32)] * 2,
    )(G, G, ea8, wc, c0, seg, gcat, bcat)


def _node_body(has_next, s0, s1, deg_ref, xe_ref, pen_ref,
               w2bd, b2, seg, ws1, bs1, ws2, bs2,
               wu1x, wu1w, bu1, gu, bu, wu2, bu2, gn, bn,
               wan, wbn, *outs):
    S = jnp.concatenate([s0[0], s1[0]], axis=-1)
    xe = xe_ref[...]
    agg = (jnp.dot(S, w2bd[...], preferred_element_type=jnp.float32)
           + deg_ref[...] * b2[...])
    t = _silu(jnp.dot(agg, ws1[...], preferred_element_type=jnp.float32) + bs1[...])
    logit = jnp.dot(t, ws2[...], preferred_element_type=jnp.float32) + bs2[...]
    aw = jax.nn.softmax(logit, axis=-1)
    weighted = agg * jnp.dot(aw, seg[...], preferred_element_type=jnp.float32)
    u = (jnp.dot(xe, wu1x[...], preferred_element_type=jnp.float32)
         + jnp.dot(weighted, wu1w[...], preferred_element_type=jnp.float32)
         + bu1[...])
    u = _silu(_ln(u, gu[...], bu[...]))
    u = jnp.dot(u, wu2[...], preferred_element_type=jnp.float32) + bu2[...]
    h = _ln(u + xe, gn[...], bn[...])
    if has_next:
        xe2_o, a0_o, a1_o, b0_o, b1_o = outs
        xe2 = h + pen_ref[...]
        xe2_o[...] = xe2
        a = jnp.dot(xe2, wan[...], preferred_element_type=jnp.float32)
        b = jnp.dot(xe2, wbn[...], preferred_element_type=jnp.float32)
        a0_o[...] = a[:, :128]
        a1_o[...] = a[:, 128:]
        b0_o[...] = b[:, :128]
        b1_o[...] = b[:, 128:]
    else:
        outs[0][...] = h


def _k_node(S2h, deg2d, xe, pe_next, W, has_next):
    grid = (N // _NB,)
    bspec_n = pl.BlockSpec((_NB, 256), lambda i: (i, 0))
    bspec_h = pl.BlockSpec((_NB, 128), lambda i: (i, 0))
    bspec_d = pl.BlockSpec((_NB, 1), lambda i: (i, 0))
    s0 = pl.BlockSpec((1, _NB, 128), lambda i: (0, i, 0))
    s1 = pl.BlockSpec((1, _NB, 128), lambda i: (1, i, 0))
    cw = lambda shape: pl.BlockSpec(shape, lambda i: tuple(0 for _ in shape))
    w_specs = [cw((256, 256)), cw((1, 256)), cw((3, 256)),
               cw((256, 64)), cw((1, 64)), cw((64, 3)), cw((1, 3)),
               cw((256, 512)), cw((256, 512)), cw((1, 512)), cw((1, 512)), cw((1, 512)),
               cw((512, 256)), cw((1, 256)), cw((1, 256)), cw((1, 256)),
               cw((256, 256)), cw((256, 256))]
    if has_next:
        out_specs = [bspec_n, bspec_h, bspec_h, bspec_h, bspec_h]
        out_shape = ([jax.ShapeDtypeStruct((N, 256), jnp.float32)]
                     + [jax.ShapeDtypeStruct((N, 128), jnp.float32)] * 4)
    else:
        out_specs = [bspec_n]
        out_shape = [jax.ShapeDtypeStruct((N, 256), jnp.float32)]
    return pl.pallas_call(
        functools.partial(_node_body, has_next),
        grid=grid,
        in_specs=[s0, s1, bspec_d, bspec_n, bspec_n] + w_specs,
        out_specs=out_specs,
        out_shape=out_shape,
    )(S2h, S2h, deg2d, xe, pe_next, *W)


def _pool_body(h_ref, oh_ref,
               wa1, ba1, wa2, ba2,
               wpf, bpf, gpf, bbpf,
               wc1, bc1, wc2, bc2,
               wz1, bz1, wz2, bz2,
               wg1, bg1, wg2, bg2,
               ws1_, bs1_, ws2_, bs2_,
               wo1, bo1, go1, bo1n, wo2, bo2, go2, bo2n, wo3, bo3,
               wt1, bt1, wt2, bt2,
               out_o, log_o, topo_o,
               msum, mxp, attn, cnt, mse):
    i = pl.program_id(0)

    @pl.when(i == 0)
    def _():
        msum[...] = jnp.zeros_like(msum)
        mxp[...] = jnp.full_like(mxp, -3e38)
        attn[...] = jnp.zeros_like(attn)
        cnt[...] = jnp.zeros_like(cnt)
        mse[...] = jnp.concatenate(
            [jnp.full((1, 1), -3e38, jnp.float32),
             jnp.zeros((1, 1), jnp.float32)], axis=-1)

    h = h_ref[...]
    oh = oh_ref[...]
    ohT_dot = lambda rhs: jax.lax.dot_general(
        oh, rhs, (((0,), (0,)), ((), ())), preferred_element_type=jnp.float32)

    s = (jnp.dot(_silu(jnp.dot(h, wa1[...], preferred_element_type=jnp.float32)
                       + ba1[...]), wa2[...],
                 preferred_element_type=jnp.float32) + ba2[...])
    m_old = mse[0, 0]
    se_old = mse[0, 1]
    m_new = jnp.maximum(m_old, jnp.max(s))
    scale = jnp.exp(m_old - m_new)
    e = jnp.exp(s - m_new)
    se_new = se_old * scale + jnp.sum(e)
    mse[...] = jnp.concatenate([jnp.full((1, 1), m_new, jnp.float32),
                                jnp.full((1, 1), se_new, jnp.float32)], axis=-1)
    attn[...] = attn[...] * scale + ohT_dot(h * e)
    msum[...] = msum[...] + ohT_dot(h)
    cnt[...] = cnt[...] + ohT_dot(jnp.ones_like(s))
    rows = [jnp.max(jnp.where(oh[:, g:g + 1] > 0.0, h, -3e38), axis=0,
                    keepdims=True) for g in range(NUM_GRAPHS)]
    mxp[...] = jnp.maximum(mxp[...], jnp.concatenate(rows, axis=0))

    mean = msum[...] / jnp.maximum(cnt[...], 1.0)
    att_pool = attn[...] / mse[0, 1]
    combined = jnp.concatenate([mean, mxp[...], att_pool], axis=-1)
    pooled = _silu(_ln(jnp.dot(combined, wpf[...],
                               preferred_element_type=jnp.float32) + bpf[...],
                       gpf[...], bbpf[...]))

    def head(w1, b1, w2, b2):
        t = _silu(jnp.dot(pooled, w1[...], preferred_element_type=jnp.float32)
                  + b1[...])
        return jnp.dot(t, w2[...], preferred_element_type=jnp.float32) + b2[...]

    chern = jnp.tanh(head(wc1, bc1, wc2, bc2))
    z2 = jax.nn.sigmoid(head(wz1, bz1, wz2, bz2))
    gap = jax.nn.softplus(head(wg1, bg1, wg2, bg2))
    sym = head(ws1_, bs1_, ws2_, bs2_)
    topo = jnp.concatenate([chern, z2, gap, sym], axis=-1)
    f = jnp.concatenate([pooled, topo], axis=-1)
    f = _silu(_ln(jnp.dot(f, wo1[...], preferred_element_type=jnp.float32)
                  + bo1[...], go1[...], bo1n[...]))
    f = _silu(_ln(jnp.dot(f, wo2[...], preferred_element_type=jnp.float32)
                  + bo2[...], go2[...], bo2n[...]))
    out = jnp.dot(f, wo3[...], preferred_element_type=jnp.float32) + bo3[...]
    t2 = _silu(jnp.dot(out, wt1[...], preferred_element_type=jnp.float32)
               + bt1[...])
    logits = jnp.dot(t2, wt2[...], preferred_element_type=jnp.float32) + bt2[...]
    out_o[...] = out
    log_o[...] = logits
    topo_o[...] = topo


def _k_pool(h, onehot, params):
    ap = params['att_pool']
    pf = params['pool_fusion']
    t = params['topo']
    op = params['output_proj']
    th = params['topo_head']
    W = [
        ap['l1']['w'], _row(ap['l1']['b']), ap['l2']['w'], _row(ap['l2']['b']),
        pf['lin']['w'], _row(pf['lin']['b']), _row(pf['ln']['g']), _row(pf['ln']['b']),
        t['chern']['l1']['w'], _row(t['chern']['l1']['b']),
        t['chern']['l2']['w'], _row(t['chern']['l2']['b']),
        t['z2']['l1']['w'], _row(t['z2']['l1']['b']),
        t['z2']['l2']['w'], _row(t['z2']['l2']['b']),
        t['gap']['l1']['w'], _row(t['gap']['l1']['b']),
        t['gap']['l2']['w'], _row(t['gap']['l2']['b']),
        t['sym']['l1']['w'], _row(t['sym']['l1']['b']),
        t['sym']['l2']['w'], _row(t['sym']['l2']['b']),
        op['l1']['w'], _row(op['l1']['b']), _row(op['ln1']['g']), _row(op['ln1']['b']),
        op['l2']['w'], _row(op['l2']['b']), _row(op['ln2']['g']), _row(op['ln2']['b']),
        op['l3']['w'], _row(op['l3']['b']),
        th['l1']['w'], _row(th['l1']['b']), th['l2']['w'], _row(th['l2']['b']),
    ]
    grid = (N // _NB,)
    bspec_n = pl.BlockSpec((_NB, 256), lambda i: (i, 0))
    bspec_o = pl.BlockSpec((_NB, 16), lambda i: (i, 0))
    cw = lambda a: pl.BlockSpec(a.shape, lambda i: tuple(0 for _ in a.shape))
    return pl.pallas_call(
        _pool_body,
        grid=grid,
        in_specs=[bspec_n, bspec_o] + [cw(w) for w in W],
        out_specs=[pl.BlockSpec((16, 128), lambda i: (0, 0)),
                   pl.BlockSpec((16, 3), lambda i: (0, 0)),
                   pl.BlockSpec((16, 14), lambda i: (0, 0))],
        out_shape=[jax.ShapeDtypeStruct((16, 128), jnp.float32),
                   jax.ShapeDtypeStruct((16, 3), jnp.float32),
                   jax.ShapeDtypeStruct((16, 14), jnp.float32)],
        scratch_shapes=[pltpu.VMEM((16, 256), jnp.float32),
                        pltpu.VMEM((16, 256), jnp.float32),
                        pltpu.VMEM((16, 256), jnp.float32),
                        pltpu.VMEM((16, 1), jnp.float32),
                        pltpu.VMEM((1, 2), jnp.float32)],
    )(h, onehot, *W)


def _layer_consts(lp):
    WAs, WBs, Wcs, c0s, gc, bc, b2c = [], [], [], [], [], [], []
    blocks = []
    for i, s in enumerate(SCALE_FACTORS):
        mp = lp['msg'][i]
        W1 = mp['l1']['w']
        WAs.append(W1[:256])
        WBs.append(W1[256:512])
        Wcs.append(W1[512:516])
        c0s.append(s * W1[516] + mp['l1']['b'])
        gc.append(mp['ln']['g'])
        bc.append(mp['ln']['b'])
        blocks.append(mp['l2']['w'])
        b2c.append(mp['l2']['b'])
    W2bd = jax.scipy.linalg.block_diag(*blocks)
    Wc = jnp.concatenate(Wcs, 1)
    Wc8 = jnp.pad(Wc, ((0, 4), (0, 0)))
    return dict(
        WA=jnp.concatenate(WAs, 1), WB=jnp.concatenate(WBs, 1),
        Wc8=Wc8, c0=jnp.concatenate(c0s),
        g=jnp.concatenate(gc), b=jnp.concatenate(bc),
        W2bd=W2bd, b2=jnp.concatenate(b2c))


def kernel(x, pos, edge_index, batch, params):
    seg = jnp.asarray(_SEG_NP)
    row = edge_index[0].astype(jnp.int32)
    col = edge_index[1].astype(jnp.int32)

    L1, L2 = params['layers'][0], params['layers'][1]
    C1, C2 = _layer_consts(L1), _layer_consts(L2)

    pos128 = jnp.pad(pos, ((0, 0), (0, 125)))
    zrows = jnp.zeros((_ROWS_PT, 128), jnp.float32)

    colp3 = col.reshape(32, _NCHP, _CHP)
    rowp3 = row.reshape(32, _NCHP, _CHP)
    evec = _sc_prologue(colp3, rowp3, pos128)
    degp = _sc_deg(colp3, zrows)
    deg2d = (degp[0, :, 0] + degp[1, :, 0])[:, None]
    ea8 = _k_ea(evec)

    pos8 = jnp.pad(pos, ((0, 0), (0, 5)))
    ip = params['input_proj']
    pe_w = lambda lp, k: lp['pos_enc'][k]
    pre_W = [
        ip['lin']['w'], _row(ip['lin']['b']), _row(ip['ln']['g']), _row(ip['ln']['b']),
        jnp.pad(pe_w(L1, 'l1')['w'], ((0, 5), (0, 0))), _row(pe_w(L1, 'l1')['b']),
        pe_w(L1, 'l2')['w'], _row(pe_w(L1, 'l2')['b']),
        jnp.pad(pe_w(L2, 'l1')['w'], ((0, 5), (0, 0))), _row(pe_w(L2, 'l1')['b']),
        pe_w(L2, 'l2')['w'], _row(pe_w(L2, 'l2')['b']),
        C1['WA'], C1['WB'],
    ]
    xe1, a0, a1, b0, b1, pe2 = _k_pre(x, pos8, pre_W)

    col3d = col.reshape(_NSUB, _NCHUNK, _CH)
    row3d = row.reshape(_NSUB, _NCHUNK, _CH)

    def edge_phase(a0, a1, b0, b1, C):
        G = _sc_gather(col3d, row3d, a0, a1, b0, b1)
        P0, P1 = _k_edge(G, ea8, C['Wc8'], _row(C['c0']), seg,
                         _row(C['g']), _row(C['b']))
        return _sc_scatter(col3d, P0, P1, zrows)

    def node_W(lp, C, Cn):
        up = lp['update']
        sa = lp['scale_att']
        Wu1 = up['l1']['w']
        return [
            C['W2bd'], _row(C['b2']), seg,
            sa['l1']['w'], _row(sa['l1']['b']), sa['l2']['w'], _row(sa['l2']['b']),
            Wu1[:256], Wu1[256:], _row(up['l1']['b']),
            _row(up['ln']['g']), _row(up['ln']['b']),
            up['l2']['w'], _row(up['l2']['b']),
            _row(lp['norm']['g']), _row(lp['norm']['b']),
            (Cn['WA'] if Cn is not None else C['WA']),
            (Cn['WB'] if Cn is not None else C['WB']),
        ]

    S1 = edge_phase(a0, a1, b0, b1, C1)
    xe2, a0, a1, b0, b1 = _k_node(S1, deg2d, xe1, pe2, node_W(L1, C1, C2), True)
    S2 = edge_phase(a0, a1, b0, b1, C2)
    (h,) = _k_node(S2, deg2d, xe2, pe2, node_W(L2, C2, None), False)

    onehot = (batch[:, None] == jnp.arange(NUM_GRAPHS)[None, :]).astype(jnp.float32)
    out, logits, topo_feat = _k_pool(h, onehot, params)
    return out, logits, topo_feat
